# baseline (speedup 1.0000x reference)
import functools
import jax
import jax.numpy as jnp
from jax.experimental import pallas as pl
from jax.experimental.pallas import tpu as pltpu

N, E, F, G, EH, BINS, L = 10000, 320000, 128, 256, 128, 40, 4
BN = 1000
BE = 512


def _silu(v):
    return v * jax.nn.sigmoid(v)


def _ln(v, s, b, eps=1e-5):
    mu = jnp.mean(v, axis=-1, keepdims=True)
    var = jnp.mean((v - mu) ** 2, axis=-1, keepdims=True)
    return (v - mu) * jax.lax.rsqrt(var + eps) * s + b


def _embed_body(x_ref, w_ref, b_ref, s_ref, bl_ref, o_ref):
    v = jnp.dot(x_ref[...], w_ref[...], preferred_element_type=jnp.float32)
    v = _silu(v + b_ref[...])
    o_ref[...] = _ln(v, s_ref[...], bl_ref[...])


def _encoder_body(xs_ref, xd_ref, r_ref, w1a_ref, w1b_ref, w1c_ref, b1_ref,
                  w2_ref, b2_ref, s_ref, bl_ref, o_ref):
    r = r_ref[...]
    dist = jnp.sqrt(jnp.sum(r * r, axis=-1, keepdims=True))
    centers = jax.lax.broadcasted_iota(jnp.int32, (1, BINS), 1).astype(jnp.float32) * (8.0 / (BINS - 1))
    width = 8.0 / BINS
    rbf = jnp.exp(-((dist - centers) ** 2) / (width ** 2))
    v = (jnp.dot(xs_ref[...], w1a_ref[...], preferred_element_type=jnp.float32)
         + jnp.dot(xd_ref[...], w1b_ref[...], preferred_element_type=jnp.float32)
         + jnp.dot(rbf, w1c_ref[...], preferred_element_type=jnp.float32))
    v = _silu(v + b1_ref[...])
    v = _silu(jnp.dot(v, w2_ref[...], preferred_element_type=jnp.float32) + b2_ref[...])
    o_ref[...] = _ln(v, s_ref[...], bl_ref[...])


def _tables_body(h_ref, wa_ref, ba_ref, wb_ref, bb_ref, wm_ref, bm_ref,
                 ws_ref, bs_ref, am_ref, bt_ref, hs_ref):
    h = h_ref[...]
    hA = jnp.dot(h, wa_ref[...], preferred_element_type=jnp.float32) + ba_ref[...]
    hB = jnp.dot(h, wb_ref[...], preferred_element_type=jnp.float32) + bb_ref[...]
    hM = jnp.dot(h, wm_ref[...], preferred_element_type=jnp.float32) + bm_ref[...]
    hS = jnp.dot(h, ws_ref[...], preferred_element_type=jnp.float32) + bs_ref[...]
    am = jnp.stack([jnp.concatenate([hA[:, 64 * p:64 * p + 64],
                                     hM[:, 64 * p:64 * p + 64]], axis=-1)
                    for p in range(4)], axis=0)
    bt = jnp.stack([hB[:, 64 * p:64 * p + 64] for p in range(4)], axis=0)
    am_ref[...] = am
    bt_ref[...] = bt
    hs_ref[...] = hS


def _ec_body(ef_ref, w_ref, b_ref, o_ref):
    o_ref[...] = jnp.dot(ef_ref[...], w_ref[...],
                         preferred_element_type=jnp.float32) + b_ref[...]


def _update_body(h_ref, hs_ref, num_ref, den_ref, s_ref, bl_ref, o_ref):
    num = num_ref[0] + num_ref[1]
    den = den_ref[0] + den_ref[1]
    v = _silu(hs_ref[...] + num / (den + 1e-6))
    o_ref[...] = _ln(v + h_ref[...], s_ref[...], bl_ref[...])


def _head_body(ef_ref, hs_ref, hd_ref, w1a_ref, w1b_ref, w1c_ref, b1_ref,
               w2_ref, b2_ref, w3_ref, b3_ref, w4_ref, b4_ref, o_ref):
    v = (jnp.dot(ef_ref[...], w1a_ref[...], preferred_element_type=jnp.float32)
         + jnp.dot(hs_ref[...], w1b_ref[...], preferred_element_type=jnp.float32)
         + jnp.dot(hd_ref[...], w1c_ref[...], preferred_element_type=jnp.float32))
    v = _silu(v + b1_ref[...])
    v = _silu(jnp.dot(v, w2_ref[...], preferred_element_type=jnp.float32) + b2_ref[...])
    v = _silu(jnp.dot(v, w3_ref[...], preferred_element_type=jnp.float32) + b3_ref[...])
    v = jnp.dot(v, w4_ref[...], preferred_element_type=jnp.float32) + b4_ref[...]
    o_ref[...] = jax.nn.sigmoid(v)


def _row_spec(blk, width):
    return pl.BlockSpec((blk, width), lambda i: (i, 0))


def _full_spec(shape):
    return pl.BlockSpec(shape, lambda i: tuple(0 for _ in shape))


def _tc_call(body, grid, in_specs, out_specs, out_shape, args):
    return pl.pallas_call(
        body, grid=(grid,), in_specs=in_specs, out_specs=out_specs,
        out_shape=out_shape)(*args)


@jax.jit
def _forward_impl(x, r, params, edge_index):
    src, dst = edge_index[0], edge_index[1]

    pe = params["atom_emb"]
    h = _tc_call(
        _embed_body, N // BN,
        [_row_spec(BN, F), _full_spec((F, G)), _full_spec((1, G)),
         _full_spec((1, G)), _full_spec((1, G))],
        _row_spec(BN, G), jax.ShapeDtypeStruct((N, G), jnp.float32),
        (x, pe["W"], pe["b"].reshape(1, G),
         params["atom_ln"]["s"].reshape(1, G), params["atom_ln"]["b"].reshape(1, G)))

    xs = x[src]
    xd = x[dst]
    r8 = jnp.pad(r, ((0, 0), (0, 5)))
    p1, p2 = params["edge1"], params["edge2"]
    ef = _tc_call(
        _encoder_body, E // BE,
        [_row_spec(BE, F), _row_spec(BE, F), _row_spec(BE, 8),
         _full_spec((F, EH)), _full_spec((F, EH)), _full_spec((BINS, EH)),
         _full_spec((1, EH)), _full_spec((EH, EH)), _full_spec((1, EH)),
         _full_spec((1, EH)), _full_spec((1, EH))],
        _row_spec(BE, EH), jax.ShapeDtypeStruct((E, EH), jnp.float32),
        (xs, xd, r8, p1["W"][:F], p1["W"][F:2 * F], p1["W"][2 * F:],
         p1["b"].reshape(1, EH), p2["W"], p2["b"].reshape(1, EH),
         params["edge_ln"]["s"].reshape(1, EH), params["edge_ln"]["b"].reshape(1, EH)))

    for lp, lnp in zip(params["gnn"], params["lns"]):
        am, bt, hs = pl.pallas_call(
            _tables_body, grid=(N // BN,),
            in_specs=[_row_spec(BN, G),
                      _full_spec((G, G)), _full_spec((1, G)),
                      _full_spec((G, G)), _full_spec((1, G)),
                      _full_spec((G, G)), _full_spec((1, G)),
                      _full_spec((G, G)), _full_spec((1, G))],
            out_specs=[pl.BlockSpec((4, BN, 128), lambda i: (0, i, 0)),
                       pl.BlockSpec((4, BN, 64), lambda i: (0, i, 0)),
                       _row_spec(BN, G)],
            out_shape=[jax.ShapeDtypeStruct((4, N, 128), jnp.float32),
                       jax.ShapeDtypeStruct((4, N, 64), jnp.float32),
                       jax.ShapeDtypeStruct((N, G), jnp.float32)],
        )(h, lp["A"]["W"], lp["A"]["b"].reshape(1, G),
          lp["B"]["W"], lp["B"]["b"].reshape(1, G),
          lp["Wm"]["W"], lp["Wm"]["b"].reshape(1, G),
          lp["Ws"]["W"], lp["Ws"]["b"].reshape(1, G))

        ec = _tc_call(
            _ec_body, E // BE,
            [_row_spec(BE, EH), _full_spec((EH, G)), _full_spec((1, G))],
            _row_spec(BE, G), jax.ShapeDtypeStruct((E, G), jnp.float32),
            (ef, lp["C"]["W"], lp["C"]["b"].reshape(1, G)))

        hA = jnp.concatenate([am[p, :, :64] for p in range(4)], axis=-1)
        hM = jnp.concatenate([am[p, :, 64:] for p in range(4)], axis=-1)
        hB = jnp.concatenate([bt[p] for p in range(4)], axis=-1)
        sigma = jax.nn.sigmoid(hA[src] + hB[dst] + ec)
        msg = sigma * hM[src]
        num = jax.ops.segment_sum(msg, dst, num_segments=N)
        den = jax.ops.segment_sum(sigma, dst, num_segments=N)
        num_parts = jnp.stack([num, jnp.zeros_like(num)], axis=0)
        den_parts = jnp.stack([den, jnp.zeros_like(den)], axis=0)

        h = _tc_call(
            _update_body, N // BN,
            [_row_spec(BN, G), _row_spec(BN, G),
             pl.BlockSpec((2, BN, G), lambda i: (0, i, 0)),
             pl.BlockSpec((2, BN, G), lambda i: (0, i, 0)),
             _full_spec((1, G)), _full_spec((1, G))],
            _row_spec(BN, G), jax.ShapeDtypeStruct((N, G), jnp.float32),
            (h, hs, num_parts, den_parts,
             lnp["s"].reshape(1, G), lnp["b"].reshape(1, G)))

    hsrc = h[src]
    hdst = h[dst]
    w1 = params["p1"]["W"]
    o = _tc_call(
        _head_body, E // BE,
        [_row_spec(BE, EH), _row_spec(BE, G), _row_spec(BE, G),
         _full_spec((EH, 256)), _full_spec((G, 256)), _full_spec((G, 256)),
         _full_spec((1, 256)), _full_spec((256, 128)), _full_spec((1, 128)),
         _full_spec((128, 64)), _full_spec((1, 64)), _full_spec((64, 1)),
         _full_spec((1, 1))],
        _row_spec(BE, 1), jax.ShapeDtypeStruct((E, 1), jnp.float32),
        (ef, hsrc, hdst, w1[:EH], w1[EH:EH + G], w1[EH + G:],
         params["p1"]["b"].reshape(1, 256),
         params["p2"]["W"], params["p2"]["b"].reshape(1, 128),
         params["p3"]["W"], params["p3"]["b"].reshape(1, 64),
         params["p4"]["W"], params["p4"]["b"].reshape(1, 1)))
    return o


def kernel(x, r, params, edge_index):
    return _forward_impl(x, r, params, edge_index)


# speedup vs baseline: 2.2086x; 2.2086x over previous
import functools
import jax
import jax.numpy as jnp
from jax import lax
from jax.experimental import pallas as pl
from jax.experimental.pallas import tpu as pltpu
from jax.experimental.pallas import tpu_sc as plsc

N, E, F, G, EH, BINS, L = 10000, 320000, 128, 256, 128, 40, 4
BN = 1000
BE = 512

NC, NS = 2, 16
NW = NC * NS
CE = 80
EPT = E // NW
NCH = EPT // CE
EPC = E // NC
NCH_MP = EPC // NS // CE
N_ACC = 10240
NRT = N_ACC // NS
NP = 4

_MESH = plsc.VectorSubcoreMesh(core_axis_name="c", subcore_axis_name="s")


def _silu(v):
    return v * jax.nn.sigmoid(v)


def _ln(v, s, b, eps=1e-5):
    mu = jnp.mean(v, axis=-1, keepdims=True)
    var = jnp.mean((v - mu) ** 2, axis=-1, keepdims=True)
    return (v - mu) * jax.lax.rsqrt(var + eps) * s + b


def _make_gather2(D):
    def body(table, idx0, idx1, out, idx_v, rows_v, sem):
        c = lax.axis_index("c")
        s = lax.axis_index("s")
        w = c * NS + s
        for k, idx in enumerate([idx0, idx1]):
            def chunk(i, _):
                base = w * EPT + i * CE
                pltpu.sync_copy(idx.at[pl.ds(base, CE)], idx_v)
                pltpu.async_copy(table.at[idx_v], rows_v, sem).wait()
                pltpu.sync_copy(rows_v, out.at[k, pl.ds(base, CE)])
                return 0
            lax.fori_loop(0, NCH, chunk, 0)

    return pl.kernel(
        body, mesh=_MESH,
        out_type=jax.ShapeDtypeStruct((2, E, D), jnp.float32),
        scratch_types=[
            pltpu.VMEM((CE,), jnp.int32),
            pltpu.VMEM((CE, D), jnp.float32),
            pltpu.SemaphoreType.DMA,
        ])


_gather2_x = _make_gather2(F)
_gather2_h = _make_gather2(G)


def _msgpass_body(am0, am1, am2, am3, bt0, bt1, ec0, ec1, ec2, ec3,
                  src_h, dst_h, zeros_h, acc_o,
                  src_v, dst_v, am_v, b_v, ec_v, ms_v,
                  acc_s, sem1, sem2):
    c = lax.axis_index("c")
    s = lax.axis_index("s")
    tile_base = c * EPC + s * (EPC // NS)
    ams = [am0, am1, am2, am3]
    bts = [bt0, bt1]
    ecs = [ec0, ec1, ec2, ec3]
    for p in range(NP):
        boff = (p % 2) * 64
        pltpu.sync_copy(zeros_h, acc_s.at[pl.ds(s * NRT, NRT)])
        plsc.subcore_barrier()

        def chunk(i, _):
            base = tile_base + i * CE
            pltpu.sync_copy(src_h.at[pl.ds(base, CE)], src_v)
            pltpu.sync_copy(dst_h.at[pl.ds(base, CE)], dst_v)
            cp1 = pltpu.async_copy(ams[p].at[src_v], am_v, sem1)
            cp2 = pltpu.async_copy(bts[p // 2].at[dst_v], b_v, sem2)
            pltpu.sync_copy(ecs[p].at[pl.ds(base, CE)], ec_v)
            cp1.wait()
            cp2.wait()

            def edge(e, _):
                for j in range(4):
                    a = am_v[e, pl.ds(16 * j, 16)]
                    m = am_v[e, pl.ds(64 + 16 * j, 16)]
                    bb = b_v[e, pl.ds(boff + 16 * j, 16)]
                    cc = ec_v[e, pl.ds(16 * j, 16)]
                    sg = 1.0 / (1.0 + jnp.exp(-(a + bb + cc)))
                    ms_v[e, pl.ds(16 * j, 16)] = sg * m
                    ms_v[e, pl.ds(64 + 16 * j, 16)] = sg
                return 0
            lax.fori_loop(0, CE, edge, 0)
            pltpu.sync_copy(ms_v, acc_s.at[dst_v], add=True)
            return 0
        lax.fori_loop(0, NCH_MP, chunk, 0)
        plsc.subcore_barrier()
        pltpu.sync_copy(acc_s.at[pl.ds(s * NRT, NRT)],
                        acc_o.at[c, p, pl.ds(s * NRT, NRT)])
        plsc.subcore_barrier()


_msgpass = pl.kernel(
    _msgpass_body, mesh=_MESH,
    out_type=jax.ShapeDtypeStruct((NC, NP, N_ACC, 128), jnp.float32),
    scratch_types=[
        pltpu.VMEM((CE,), jnp.int32),
        pltpu.VMEM((CE,), jnp.int32),
        pltpu.VMEM((CE, 128), jnp.float32),
        pltpu.VMEM((CE, 128), jnp.float32),
        pltpu.VMEM((CE, 64), jnp.float32),
        pltpu.VMEM((CE, 128), jnp.float32),
        pltpu.VMEM_SHARED((N_ACC, 128), jnp.float32),
        pltpu.SemaphoreType.DMA,
        pltpu.SemaphoreType.DMA,
    ])


def _embed_body(x_ref, w_ref, b_ref, s_ref, bl_ref, o_ref):
    v = jnp.dot(x_ref[...], w_ref[...], preferred_element_type=jnp.float32)
    v = _silu(v + b_ref[...])
    o_ref[...] = _ln(v, s_ref[...], bl_ref[...])


def _encoder_body(xg0_ref, xg1_ref, r_ref, w1a_ref, w1b_ref, w1c_ref, b1_ref,
                  w2_ref, b2_ref, s_ref, bl_ref, o_ref):
    r = r_ref[...]
    dist = jnp.sqrt(jnp.sum(r * r, axis=-1, keepdims=True))
    centers = jax.lax.broadcasted_iota(jnp.int32, (1, BINS), 1).astype(jnp.float32) * (8.0 / (BINS - 1))
    width = 8.0 / BINS
    rbf = jnp.exp(-((dist - centers) ** 2) / (width ** 2))
    v = (jnp.dot(xg0_ref[0], w1a_ref[...], preferred_element_type=jnp.float32)
         + jnp.dot(xg1_ref[0], w1b_ref[...], preferred_element_type=jnp.float32)
         + jnp.dot(rbf, w1c_ref[...], preferred_element_type=jnp.float32))
    v = _silu(v + b1_ref[...])
    v = _silu(jnp.dot(v, w2_ref[...], preferred_element_type=jnp.float32) + b2_ref[...])
    o_ref[...] = _ln(v, s_ref[...], bl_ref[...])


def _tables_body(h_ref, wa_ref, ba_ref, wb_ref, bb_ref, wm_ref, bm_ref,
                 ws_ref, bs_ref, am0_ref, am1_ref, am2_ref, am3_ref,
                 bt0_ref, bt1_ref, hs_ref):
    h = h_ref[...]
    hA = jnp.dot(h, wa_ref[...], preferred_element_type=jnp.float32) + ba_ref[...]
    hB = jnp.dot(h, wb_ref[...], preferred_element_type=jnp.float32) + bb_ref[...]
    hM = jnp.dot(h, wm_ref[...], preferred_element_type=jnp.float32) + bm_ref[...]
    hS = jnp.dot(h, ws_ref[...], preferred_element_type=jnp.float32) + bs_ref[...]
    for p, am_ref in enumerate([am0_ref, am1_ref, am2_ref, am3_ref]):
        am_ref[...] = jnp.concatenate(
            [hA[:, 64 * p:64 * p + 64], hM[:, 64 * p:64 * p + 64]], axis=-1)
    bt0_ref[...] = hB[:, :128]
    bt1_ref[...] = hB[:, 128:]
    hs_ref[...] = hS


def _ec_body(ef_ref, w_ref, b_ref, e0_ref, e1_ref, e2_ref, e3_ref):
    v = jnp.dot(ef_ref[...], w_ref[...],
                preferred_element_type=jnp.float32) + b_ref[...]
    for p, o_ref in enumerate([e0_ref, e1_ref, e2_ref, e3_ref]):
        o_ref[...] = v[:, 64 * p:64 * p + 64]


def _update_body(h_ref, hs_ref, acc_ref, s_ref, bl_ref, o_ref):
    num = jnp.concatenate(
        [acc_ref[0, p, :, :64] + acc_ref[1, p, :, :64] for p in range(NP)], axis=-1)
    den = jnp.concatenate(
        [acc_ref[0, p, :, 64:] + acc_ref[1, p, :, 64:] for p in range(NP)], axis=-1)
    v = _silu(hs_ref[...] + num / (den + 1e-6))
    o_ref[...] = _ln(v + h_ref[...], s_ref[...], bl_ref[...])


def _head_body(ef_ref, hs_ref, hd_ref, w1a_ref, w1b_ref, w1c_ref, b1_ref,
               w2_ref, b2_ref, w3_ref, b3_ref, w4_ref, b4_ref, o_ref):
    v = (jnp.dot(ef_ref[...], w1a_ref[...], preferred_element_type=jnp.float32)
         + jnp.dot(hs_ref[0], w1b_ref[...], preferred_element_type=jnp.float32)
         + jnp.dot(hd_ref[0], w1c_ref[...], preferred_element_type=jnp.float32))
    v = _silu(v + b1_ref[...])
    v = _silu(jnp.dot(v, w2_ref[...], preferred_element_type=jnp.float32) + b2_ref[...])
    v = _silu(jnp.dot(v, w3_ref[...], preferred_element_type=jnp.float32) + b3_ref[...])
    v = jnp.dot(v, w4_ref[...], preferred_element_type=jnp.float32) + b4_ref[...]
    o_ref[...] = jax.nn.sigmoid(v)


def _row_spec(blk, width):
    return pl.BlockSpec((blk, width), lambda i: (i, 0))


def _full_spec(shape):
    return pl.BlockSpec(shape, lambda i: tuple(0 for _ in shape))


def _tc_call(body, grid, in_specs, out_specs, out_shape, args):
    return pl.pallas_call(
        body, grid=(grid,), in_specs=in_specs, out_specs=out_specs,
        out_shape=out_shape)(*args)


@jax.jit
def _forward_impl(x, r, params, edge_index):
    pe = params["atom_emb"]
    h = _tc_call(
        _embed_body, N // BN,
        [_row_spec(BN, F), _full_spec((F, G)), _full_spec((1, G)),
         _full_spec((1, G)), _full_spec((1, G))],
        _row_spec(BN, G), jax.ShapeDtypeStruct((N, G), jnp.float32),
        (x, pe["W"], pe["b"].reshape(1, G),
         params["atom_ln"]["s"].reshape(1, G), params["atom_ln"]["b"].reshape(1, G)))

    src = edge_index[0]
    dst = edge_index[1]
    xg = _gather2_x(x, src, dst)
    r8 = jnp.pad(r, ((0, 0), (0, 5)))
    p1, p2 = params["edge1"], params["edge2"]
    ef = _tc_call(
        _encoder_body, E // BE,
        [pl.BlockSpec((1, BE, F), lambda i: (0, i, 0)),
         pl.BlockSpec((1, BE, F), lambda i: (1, i, 0)),
         _row_spec(BE, 8),
         _full_spec((F, EH)), _full_spec((F, EH)), _full_spec((BINS, EH)),
         _full_spec((1, EH)), _full_spec((EH, EH)), _full_spec((1, EH)),
         _full_spec((1, EH)), _full_spec((1, EH))],
        _row_spec(BE, EH), jax.ShapeDtypeStruct((E, EH), jnp.float32),
        (xg, xg, r8, p1["W"][:F], p1["W"][F:2 * F], p1["W"][2 * F:],
         p1["b"].reshape(1, EH), p2["W"], p2["b"].reshape(1, EH),
         params["edge_ln"]["s"].reshape(1, EH), params["edge_ln"]["b"].reshape(1, EH)))

    zeros_h = jnp.zeros((NRT, 128), jnp.float32)

    for lp, lnp in zip(params["gnn"], params["lns"]):
        tbl = pl.pallas_call(
            _tables_body, grid=(N // BN,),
            in_specs=[_row_spec(BN, G),
                      _full_spec((G, G)), _full_spec((1, G)),
                      _full_spec((G, G)), _full_spec((1, G)),
                      _full_spec((G, G)), _full_spec((1, G)),
                      _full_spec((G, G)), _full_spec((1, G))],
            out_specs=[_row_spec(BN, 128)] * 6 + [_row_spec(BN, G)],
            out_shape=[jax.ShapeDtypeStruct((N, 128), jnp.float32)] * 6
                      + [jax.ShapeDtypeStruct((N, G), jnp.float32)],
        )(h, lp["A"]["W"], lp["A"]["b"].reshape(1, G),
          lp["B"]["W"], lp["B"]["b"].reshape(1, G),
          lp["Wm"]["W"], lp["Wm"]["b"].reshape(1, G),
          lp["Ws"]["W"], lp["Ws"]["b"].reshape(1, G))
        am = tbl[:4]
        bt = tbl[4:6]
        hs = tbl[6]

        ec = _tc_call(
            _ec_body, E // BE,
            [_row_spec(BE, EH), _full_spec((EH, G)), _full_spec((1, G))],
            [_row_spec(BE, 64)] * 4,
            [jax.ShapeDtypeStruct((E, 64), jnp.float32)] * 4,
            (ef, lp["C"]["W"], lp["C"]["b"].reshape(1, G)))

        acc = _msgpass(*am, *bt, *ec, src, dst, zeros_h)

        h = _tc_call(
            _update_body, N // BN,
            [_row_spec(BN, G), _row_spec(BN, G),
             pl.BlockSpec((NC, NP, BN, 128), lambda i: (0, 0, i, 0)),
             _full_spec((1, G)), _full_spec((1, G))],
            _row_spec(BN, G), jax.ShapeDtypeStruct((N, G), jnp.float32),
            (h, hs, acc,
             lnp["s"].reshape(1, G), lnp["b"].reshape(1, G)))

    hg = _gather2_h(h, src, dst)
    w1 = params["p1"]["W"]
    o = _tc_call(
        _head_body, E // BE,
        [_row_spec(BE, EH),
         pl.BlockSpec((1, BE, G), lambda i: (0, i, 0)),
         pl.BlockSpec((1, BE, G), lambda i: (1, i, 0)),
         _full_spec((EH, 256)), _full_spec((G, 256)), _full_spec((G, 256)),
         _full_spec((1, 256)), _full_spec((256, 128)), _full_spec((1, 128)),
         _full_spec((128, 64)), _full_spec((1, 64)), _full_spec((64, 1)),
         _full_spec((1, 1))],
        _row_spec(BE, 1), jax.ShapeDtypeStruct((E, 1), jnp.float32),
        (ef, hg, hg, w1[:EH], w1[EH:EH + G], w1[EH + G:],
         params["p1"]["b"].reshape(1, 256),
         params["p2"]["W"], params["p2"]["b"].reshape(1, 128),
         params["p3"]["W"], params["p3"]["b"].reshape(1, 64),
         params["p4"]["W"], params["p4"]["b"].reshape(1, 1)))
    return o


def kernel(x, r, params, edge_index):
    return _forward_impl(x, r, params, edge_index)


# speedup vs baseline: 3.0741x; 1.3919x over previous
import functools
import jax
import jax.numpy as jnp
from jax import lax
from jax.experimental import pallas as pl
from jax.experimental.pallas import tpu as pltpu
from jax.experimental.pallas import tpu_sc as plsc

N, E, F, G, EH, BINS, L = 10000, 320000, 128, 256, 128, 40, 4
BN = 1000
BE = 512

NC, NS = 2, 16
NW = NC * NS
CE = 80
EPT = E // NW
NCH = EPT // CE
EPC = E // NC
CM = 40
NCH_MP = EPC // NS // CM
N_ACC = 10240
NRT = N_ACC // NS
NP = 4

_MESH = plsc.VectorSubcoreMesh(core_axis_name="c", subcore_axis_name="s")


def _silu(v):
    return v * jax.nn.sigmoid(v)


def _ln(v, s, b, eps=1e-5):
    mu = jnp.mean(v, axis=-1, keepdims=True)
    var = jnp.mean((v - mu) ** 2, axis=-1, keepdims=True)
    return (v - mu) * jax.lax.rsqrt(var + eps) * s + b


def _make_gather2(D):
    def body(table, idx0, idx1, out, idx_v, rows_v, si0, si1, sg0, sg1):
        c = lax.axis_index("c")
        s = lax.axis_index("s")
        w = c * NS + s
        sidx = [si0, si1]
        sgat = [sg0, sg1]
        for k, idx in enumerate([idx0, idx1]):
            base0 = w * EPT

            def islice(ch):
                return idx.at[pl.ds(base0 + ch * CE, CE)]

            pltpu.sync_copy(islice(0), idx_v.at[0])
            pltpu.async_copy(table.at[idx_v.at[0]], rows_v.at[0], sgat[0])
            pltpu.async_copy(islice(1), idx_v.at[1], sidx[1])

            def step(i, _):
                for b in range(2):
                    ch = 2 * i + b
                    nxt = 1 - b
                    pltpu.make_async_copy(islice(ch + 1), idx_v.at[nxt],
                                          sidx[nxt]).wait()
                    pltpu.async_copy(table.at[idx_v.at[nxt]],
                                     rows_v.at[nxt], sgat[nxt])
                    pltpu.make_async_copy(table.at[idx_v.at[b]],
                                          rows_v.at[b], sgat[b]).wait()
                    pltpu.sync_copy(rows_v.at[b],
                                    out.at[k, pl.ds(base0 + ch * CE, CE)])

                    @pl.when(ch + 2 < NCH)
                    def _():
                        pltpu.async_copy(islice(ch + 2), idx_v.at[b], sidx[b])
                return 0
            lax.fori_loop(0, (NCH - 1) // 2, step, 0)
            pltpu.make_async_copy(table.at[idx_v.at[0]], rows_v.at[0],
                                  sgat[0]).wait()
            pltpu.sync_copy(rows_v.at[0],
                            out.at[k, pl.ds(base0 + (NCH - 1) * CE, CE)])

    return pl.kernel(
        body, mesh=_MESH,
        out_type=jax.ShapeDtypeStruct((2, E, D), jnp.float32),
        scratch_types=[
            pltpu.VMEM((2, CE), jnp.int32),
            pltpu.VMEM((2, CE, D), jnp.float32),
            pltpu.SemaphoreType.DMA,
            pltpu.SemaphoreType.DMA,
            pltpu.SemaphoreType.DMA,
            pltpu.SemaphoreType.DMA,
        ])


_gather2_x = _make_gather2(F)
_gather2_h = _make_gather2(G)


def _msgpass_body(am0, am1, am2, am3, bt0, bt1, ec0, ec1, ec2, ec3,
                  src_h, dst_h, zeros_h, acc_o,
                  src_v, dst_v, am_v, b_v, ec_v, ms_v, acc_s,
                  si0, si1, sa0, sa1, sb0, sb1, se0, se1):
    c = lax.axis_index("c")
    s = lax.axis_index("s")
    tile_base = c * EPC + s * (EPC // NS)
    ams = [am0, am1, am2, am3]
    bts = [bt0, bt1]
    ecs = [ec0, ec1, ec2, ec3]
    sidx = [si0, si1]
    sam = [sa0, sa1]
    sbt = [sb0, sb1]
    sec = [se0, se1]
    for p in range(NP):
        boff = (p % 2) * 64
        amt = ams[p]
        btt = bts[p // 2]
        ect = ecs[p]

        def sslice(ch):
            return src_h.at[pl.ds(tile_base + ch * CM, CM)]

        def dslice(ch):
            return dst_h.at[pl.ds(tile_base + ch * CM, CM)]

        def eslice(ch):
            return ect.at[pl.ds(tile_base + ch * CM, CM)]

        def launch_gathers(ch, b):
            pltpu.async_copy(amt.at[src_v.at[b]], am_v.at[b], sam[b])
            pltpu.async_copy(btt.at[dst_v.at[b]], b_v.at[b], sbt[b])
            pltpu.async_copy(eslice(ch), ec_v.at[b], sec[b])

        def wait_gathers(ch, b):
            pltpu.make_async_copy(amt.at[src_v.at[b]], am_v.at[b],
                                  sam[b]).wait()
            pltpu.make_async_copy(btt.at[dst_v.at[b]], b_v.at[b],
                                  sbt[b]).wait()
            pltpu.make_async_copy(eslice(ch), ec_v.at[b], sec[b]).wait()

        def compute_scatter(b):
            amb = am_v.at[b]
            bvb = b_v.at[b]
            ecb = ec_v.at[b]
            msb = ms_v.at[b]

            def edge(e, _):
                for j in range(4):
                    a = amb[e, pl.ds(16 * j, 16)]
                    m = amb[e, pl.ds(64 + 16 * j, 16)]
                    bb = bvb[e, pl.ds(boff + 16 * j, 16)]
                    cc = ecb[e, pl.ds(16 * j, 16)]
                    sg = 1.0 / (1.0 + jnp.exp(-(a + bb + cc)))
                    msb[e, pl.ds(16 * j, 16)] = sg * m
                    msb[e, pl.ds(64 + 16 * j, 16)] = sg
                return 0
            lax.fori_loop(0, CM, edge, 0)
            pltpu.sync_copy(msb, acc_s.at[dst_v.at[b]], add=True)

        pltpu.sync_copy(zeros_h, acc_s.at[pl.ds(s * NRT, NRT)])
        plsc.subcore_barrier()

        pltpu.sync_copy(sslice(0), src_v.at[0])
        pltpu.sync_copy(dslice(0), dst_v.at[0])
        launch_gathers(0, 0)
        pltpu.async_copy(sslice(1), src_v.at[1], sidx[1])
        pltpu.async_copy(dslice(1), dst_v.at[1], sidx[1])

        def step(i, _):
            for b in range(2):
                ch = 2 * i + b
                nxt = 1 - b

                @pl.when(ch + 1 < NCH_MP)
                def _():
                    pltpu.make_async_copy(sslice(ch + 1), src_v.at[nxt],
                                          sidx[nxt]).wait()
                    pltpu.make_async_copy(dslice(ch + 1), dst_v.at[nxt],
                                          sidx[nxt]).wait()
                    launch_gathers(ch + 1, nxt)

                wait_gathers(ch, b)
                compute_scatter(b)

                @pl.when(ch + 2 < NCH_MP)
                def _():
                    pltpu.async_copy(sslice(ch + 2), src_v.at[b], sidx[b])
                    pltpu.async_copy(dslice(ch + 2), dst_v.at[b], sidx[b])
            return 0
        lax.fori_loop(0, NCH_MP // 2, step, 0)

        plsc.subcore_barrier()
        pltpu.sync_copy(acc_s.at[pl.ds(s * NRT, NRT)],
                        acc_o.at[c, p, pl.ds(s * NRT, NRT)])
        plsc.subcore_barrier()


_msgpass = pl.kernel(
    _msgpass_body, mesh=_MESH,
    out_type=jax.ShapeDtypeStruct((NC, NP, N_ACC, 128), jnp.float32),
    scratch_types=[
        pltpu.VMEM((2, CM), jnp.int32),
        pltpu.VMEM((2, CM), jnp.int32),
        pltpu.VMEM((2, CM, 128), jnp.float32),
        pltpu.VMEM((2, CM, 128), jnp.float32),
        pltpu.VMEM((2, CM, 64), jnp.float32),
        pltpu.VMEM((2, CM, 128), jnp.float32),
        pltpu.VMEM_SHARED((N_ACC, 128), jnp.float32),
        pltpu.SemaphoreType.DMA,
        pltpu.SemaphoreType.DMA,
        pltpu.SemaphoreType.DMA,
        pltpu.SemaphoreType.DMA,
        pltpu.SemaphoreType.DMA,
        pltpu.SemaphoreType.DMA,
        pltpu.SemaphoreType.DMA,
        pltpu.SemaphoreType.DMA,
    ])


def _embed_body(x_ref, w_ref, b_ref, s_ref, bl_ref, o_ref):
    v = jnp.dot(x_ref[...], w_ref[...], preferred_element_type=jnp.float32)
    v = _silu(v + b_ref[...])
    o_ref[...] = _ln(v, s_ref[...], bl_ref[...])


def _encoder_body(xg0_ref, xg1_ref, r_ref, w1a_ref, w1b_ref, w1c_ref, b1_ref,
                  w2_ref, b2_ref, s_ref, bl_ref, o_ref):
    r = r_ref[...]
    dist = jnp.sqrt(jnp.sum(r * r, axis=-1, keepdims=True))
    centers = jax.lax.broadcasted_iota(jnp.int32, (1, BINS), 1).astype(jnp.float32) * (8.0 / (BINS - 1))
    width = 8.0 / BINS
    rbf = jnp.exp(-((dist - centers) ** 2) / (width ** 2))
    v = (jnp.dot(xg0_ref[0], w1a_ref[...], preferred_element_type=jnp.float32)
         + jnp.dot(xg1_ref[0], w1b_ref[...], preferred_element_type=jnp.float32)
         + jnp.dot(rbf, w1c_ref[...], preferred_element_type=jnp.float32))
    v = _silu(v + b1_ref[...])
    v = _silu(jnp.dot(v, w2_ref[...], preferred_element_type=jnp.float32) + b2_ref[...])
    o_ref[...] = _ln(v, s_ref[...], bl_ref[...])


def _tables_body(h_ref, wa_ref, ba_ref, wb_ref, bb_ref, wm_ref, bm_ref,
                 ws_ref, bs_ref, am0_ref, am1_ref, am2_ref, am3_ref,
                 bt0_ref, bt1_ref, hs_ref):
    h = h_ref[...]
    hA = jnp.dot(h, wa_ref[...], preferred_element_type=jnp.float32) + ba_ref[...]
    hB = jnp.dot(h, wb_ref[...], preferred_element_type=jnp.float32) + bb_ref[...]
    hM = jnp.dot(h, wm_ref[...], preferred_element_type=jnp.float32) + bm_ref[...]
    hS = jnp.dot(h, ws_ref[...], preferred_element_type=jnp.float32) + bs_ref[...]
    for p, am_ref in enumerate([am0_ref, am1_ref, am2_ref, am3_ref]):
        am_ref[...] = jnp.concatenate(
            [hA[:, 64 * p:64 * p + 64], hM[:, 64 * p:64 * p + 64]], axis=-1)
    bt0_ref[...] = hB[:, :128]
    bt1_ref[...] = hB[:, 128:]
    hs_ref[...] = hS


def _ec_body(ef_ref, w_ref, b_ref, e0_ref, e1_ref, e2_ref, e3_ref):
    v = jnp.dot(ef_ref[...], w_ref[...],
                preferred_element_type=jnp.float32) + b_ref[...]
    for p, o_ref in enumerate([e0_ref, e1_ref, e2_ref, e3_ref]):
        o_ref[...] = v[:, 64 * p:64 * p + 64]


def _update_body(h_ref, hs_ref, acc_ref, s_ref, bl_ref, o_ref):
    num = jnp.concatenate(
        [acc_ref[0, p, :, :64] + acc_ref[1, p, :, :64] for p in range(NP)], axis=-1)
    den = jnp.concatenate(
        [acc_ref[0, p, :, 64:] + acc_ref[1, p, :, 64:] for p in range(NP)], axis=-1)
    v = _silu(hs_ref[...] + num / (den + 1e-6))
    o_ref[...] = _ln(v + h_ref[...], s_ref[...], bl_ref[...])


def _head_body(ef_ref, hs_ref, hd_ref, w1a_ref, w1b_ref, w1c_ref, b1_ref,
               w2_ref, b2_ref, w3_ref, b3_ref, w4_ref, b4_ref, o_ref):
    v = (jnp.dot(ef_ref[...], w1a_ref[...], preferred_element_type=jnp.float32)
         + jnp.dot(hs_ref[0], w1b_ref[...], preferred_element_type=jnp.float32)
         + jnp.dot(hd_ref[0], w1c_ref[...], preferred_element_type=jnp.float32))
    v = _silu(v + b1_ref[...])
    v = _silu(jnp.dot(v, w2_ref[...], preferred_element_type=jnp.float32) + b2_ref[...])
    v = _silu(jnp.dot(v, w3_ref[...], preferred_element_type=jnp.float32) + b3_ref[...])
    v = jnp.dot(v, w4_ref[...], preferred_element_type=jnp.float32) + b4_ref[...]
    o_ref[...] = jax.nn.sigmoid(v)


def _row_spec(blk, width):
    return pl.BlockSpec((blk, width), lambda i: (i, 0))


def _full_spec(shape):
    return pl.BlockSpec(shape, lambda i: tuple(0 for _ in shape))


def _tc_call(body, grid, in_specs, out_specs, out_shape, args):
    return pl.pallas_call(
        body, grid=(grid,), in_specs=in_specs, out_specs=out_specs,
        out_shape=out_shape)(*args)


@jax.jit
def _forward_impl(x, r, params, edge_index):
    pe = params["atom_emb"]
    h = _tc_call(
        _embed_body, N // BN,
        [_row_spec(BN, F), _full_spec((F, G)), _full_spec((1, G)),
         _full_spec((1, G)), _full_spec((1, G))],
        _row_spec(BN, G), jax.ShapeDtypeStruct((N, G), jnp.float32),
        (x, pe["W"], pe["b"].reshape(1, G),
         params["atom_ln"]["s"].reshape(1, G), params["atom_ln"]["b"].reshape(1, G)))

    src = edge_index[0]
    dst = edge_index[1]
    xg = _gather2_x(x, src, dst)
    r8 = jnp.pad(r, ((0, 0), (0, 5)))
    p1, p2 = params["edge1"], params["edge2"]
    ef = _tc_call(
        _encoder_body, E // BE,
        [pl.BlockSpec((1, BE, F), lambda i: (0, i, 0)),
         pl.BlockSpec((1, BE, F), lambda i: (1, i, 0)),
         _row_spec(BE, 8),
         _full_spec((F, EH)), _full_spec((F, EH)), _full_spec((BINS, EH)),
         _full_spec((1, EH)), _full_spec((EH, EH)), _full_spec((1, EH)),
         _full_spec((1, EH)), _full_spec((1, EH))],
        _row_spec(BE, EH), jax.ShapeDtypeStruct((E, EH), jnp.float32),
        (xg, xg, r8, p1["W"][:F], p1["W"][F:2 * F], p1["W"][2 * F:],
         p1["b"].reshape(1, EH), p2["W"], p2["b"].reshape(1, EH),
         params["edge_ln"]["s"].reshape(1, EH), params["edge_ln"]["b"].reshape(1, EH)))

    zeros_h = jnp.zeros((NRT, 128), jnp.float32)

    for lp, lnp in zip(params["gnn"], params["lns"]):
        tbl = pl.pallas_call(
            _tables_body, grid=(N // BN,),
            in_specs=[_row_spec(BN, G),
                      _full_spec((G, G)), _full_spec((1, G)),
                      _full_spec((G, G)), _full_spec((1, G)),
                      _full_spec((G, G)), _full_spec((1, G)),
                      _full_spec((G, G)), _full_spec((1, G))],
            out_specs=[_row_spec(BN, 128)] * 6 + [_row_spec(BN, G)],
            out_shape=[jax.ShapeDtypeStruct((N, 128), jnp.float32)] * 6
                      + [jax.ShapeDtypeStruct((N, G), jnp.float32)],
        )(h, lp["A"]["W"], lp["A"]["b"].reshape(1, G),
          lp["B"]["W"], lp["B"]["b"].reshape(1, G),
          lp["Wm"]["W"], lp["Wm"]["b"].reshape(1, G),
          lp["Ws"]["W"], lp["Ws"]["b"].reshape(1, G))
        am = tbl[:4]
        bt = tbl[4:6]
        hs = tbl[6]

        ec = _tc_call(
            _ec_body, E // BE,
            [_row_spec(BE, EH), _full_spec((EH, G)), _full_spec((1, G))],
            [_row_spec(BE, 64)] * 4,
            [jax.ShapeDtypeStruct((E, 64), jnp.float32)] * 4,
            (ef, lp["C"]["W"], lp["C"]["b"].reshape(1, G)))

        acc = _msgpass(*am, *bt, *ec, src, dst, zeros_h)

        h = _tc_call(
            _update_body, N // BN,
            [_row_spec(BN, G), _row_spec(BN, G),
             pl.BlockSpec((NC, NP, BN, 128), lambda i: (0, 0, i, 0)),
             _full_spec((1, G)), _full_spec((1, G))],
            _row_spec(BN, G), jax.ShapeDtypeStruct((N, G), jnp.float32),
            (h, hs, acc,
             lnp["s"].reshape(1, G), lnp["b"].reshape(1, G)))

    hg = _gather2_h(h, src, dst)
    w1 = params["p1"]["W"]
    o = _tc_call(
        _head_body, E // BE,
        [_row_spec(BE, EH),
         pl.BlockSpec((1, BE, G), lambda i: (0, i, 0)),
         pl.BlockSpec((1, BE, G), lambda i: (1, i, 0)),
         _full_spec((EH, 256)), _full_spec((G, 256)), _full_spec((G, 256)),
         _full_spec((1, 256)), _full_spec((256, 128)), _full_spec((1, 128)),
         _full_spec((128, 64)), _full_spec((1, 64)), _full_spec((64, 1)),
         _full_spec((1, 1))],
        _row_spec(BE, 1), jax.ShapeDtypeStruct((E, 1), jnp.float32),
        (ef, hg, hg, w1[:EH], w1[EH:EH + G], w1[EH + G:],
         params["p1"]["b"].reshape(1, 256),
         params["p2"]["W"], params["p2"]["b"].reshape(1, 128),
         params["p3"]["W"], params["p3"]["b"].reshape(1, 64),
         params["p4"]["W"], params["p4"]["b"].reshape(1, 1)))
    return o


def kernel(x, r, params, edge_index):
    return _forward_impl(x, r, params, edge_index)


# speedup vs baseline: 3.1174x; 1.0141x over previous
import functools
import numpy as np
import jax
import jax.numpy as jnp
from jax import lax
from jax.experimental import pallas as pl
from jax.experimental.pallas import tpu as pltpu
from jax.experimental.pallas import tpu_sc as plsc

N, E, F, G, EH, BINS, L = 10000, 320000, 128, 256, 128, 40, 4
BN = 1000
BE = 512

NC, NS = 2, 16
NW = NC * NS
CE = 80
EPT = E // NW
NCH = EPT // CE
EPC = E // NC
CM = 40
NCH_MP = EPC // NS // CM
N_ACC = 10240
NRT = N_ACC // NS
NP = 4

_MESH = plsc.VectorSubcoreMesh(core_axis_name="c", subcore_axis_name="s")


def _silu(v):
    return v * jax.nn.sigmoid(v)


def _ln(v, s, b, eps=1e-5):
    mu = jnp.mean(v, axis=-1, keepdims=True)
    var = jnp.mean((v - mu) ** 2, axis=-1, keepdims=True)
    return (v - mu) * jax.lax.rsqrt(var + eps) * s + b


def _make_gather2(D, dtype=jnp.float32, split=1):
    rshape = (CE, D) if split == 1 else (CE, split, D // split)
    oshape = (2, E, D) if split == 1 else (2, E, split, D // split)

    def body(table, idx0, idx1, out, idx_v, rows_v, si0, si1, sg0, sg1):
        c = lax.axis_index("c")
        s = lax.axis_index("s")
        w = c * NS + s
        sidx = [si0, si1]
        sgat = [sg0, sg1]
        for k, idx in enumerate([idx0, idx1]):
            base0 = w * EPT

            def islice(ch):
                return idx.at[pl.ds(base0 + ch * CE, CE)]

            pltpu.sync_copy(islice(0), idx_v.at[0])
            pltpu.async_copy(table.at[idx_v.at[0]], rows_v.at[0], sgat[0])
            pltpu.async_copy(islice(1), idx_v.at[1], sidx[1])

            def step(i, _):
                for b in range(2):
                    ch = 2 * i + b
                    nxt = 1 - b
                    pltpu.make_async_copy(islice(ch + 1), idx_v.at[nxt],
                                          sidx[nxt]).wait()
                    pltpu.async_copy(table.at[idx_v.at[nxt]],
                                     rows_v.at[nxt], sgat[nxt])
                    pltpu.make_async_copy(table.at[idx_v.at[b]],
                                          rows_v.at[b], sgat[b]).wait()
                    pltpu.sync_copy(rows_v.at[b],
                                    out.at[k, pl.ds(base0 + ch * CE, CE)])

                    @pl.when(ch + 2 < NCH)
                    def _():
                        pltpu.async_copy(islice(ch + 2), idx_v.at[b], sidx[b])
                return 0
            lax.fori_loop(0, (NCH - 1) // 2, step, 0)
            pltpu.make_async_copy(table.at[idx_v.at[0]], rows_v.at[0],
                                  sgat[0]).wait()
            pltpu.sync_copy(rows_v.at[0],
                            out.at[k, pl.ds(base0 + (NCH - 1) * CE, CE)])

    return pl.kernel(
        body, mesh=_MESH,
        out_type=jax.ShapeDtypeStruct(oshape, dtype),
        scratch_types=[
            pltpu.VMEM((2, CE), jnp.int32),
            pltpu.VMEM((2,) + rshape, dtype),
            pltpu.SemaphoreType.DMA,
            pltpu.SemaphoreType.DMA,
            pltpu.SemaphoreType.DMA,
            pltpu.SemaphoreType.DMA,
        ])


_gather2_x = _make_gather2(F)
_gather2_h = _make_gather2(G // 2, jnp.int32)


def _msgpass_body(am0, am1, am2, am3, bt0, bt1, ec0, ec1, ec2, ec3,
                  src_h, dst_h, zeros_h, acc_o,
                  src_v, dst_v, am_v, b_v, ec_v, ms_v, acc_s,
                  si0, si1, sa0, sa1, sb0, sb1, se0, se1):
    c = lax.axis_index("c")
    s = lax.axis_index("s")
    tile_base = c * EPC + s * (EPC // NS)
    ams = [am0, am1, am2, am3]
    bts = [bt0, bt1]
    ecs = [ec0, ec1, ec2, ec3]
    sidx = [si0, si1]
    sam = [sa0, sa1]
    sbt = [sb0, sb1]
    sec = [se0, se1]
    for p in range(NP):
        boff = (p % 2) * 64
        amt = ams[p]
        btt = bts[p // 2]
        ect = ecs[p]

        def sslice(ch):
            return src_h.at[pl.ds(tile_base + ch * CM, CM)]

        def dslice(ch):
            return dst_h.at[pl.ds(tile_base + ch * CM, CM)]

        def eslice(ch):
            return ect.at[pl.ds(tile_base + ch * CM, CM)]

        def launch_gathers(ch, b):
            pltpu.async_copy(amt.at[src_v.at[b]], am_v.at[b], sam[b])
            pltpu.async_copy(btt.at[dst_v.at[b]], b_v.at[b], sbt[b])
            pltpu.async_copy(eslice(ch), ec_v.at[b], sec[b])

        def wait_gathers(ch, b):
            pltpu.make_async_copy(amt.at[src_v.at[b]], am_v.at[b],
                                  sam[b]).wait()
            pltpu.make_async_copy(btt.at[dst_v.at[b]], b_v.at[b],
                                  sbt[b]).wait()
            pltpu.make_async_copy(eslice(ch), ec_v.at[b], sec[b]).wait()

        def compute_scatter(b):
            amb = am_v.at[b]
            bvb = b_v.at[b]
            ecb = ec_v.at[b]
            msb = ms_v.at[b]

            def edge(e, _):
                for j in range(4):
                    a = amb[e, pl.ds(16 * j, 16)]
                    m = amb[e, pl.ds(64 + 16 * j, 16)]
                    bb = bvb[e, pl.ds(boff + 16 * j, 16)]
                    cc = ecb[e, pl.ds(16 * j, 16)]
                    sg = 1.0 / (1.0 + jnp.exp(-(a + bb + cc)))
                    msb[e, pl.ds(16 * j, 16)] = sg * m
                    msb[e, pl.ds(64 + 16 * j, 16)] = sg
                return 0
            lax.fori_loop(0, CM, edge, 0)
            pltpu.sync_copy(msb, acc_s.at[dst_v.at[b]], add=True)

        pltpu.sync_copy(zeros_h, acc_s.at[pl.ds(s * NRT, NRT)])
        plsc.subcore_barrier()

        pltpu.sync_copy(sslice(0), src_v.at[0])
        pltpu.sync_copy(dslice(0), dst_v.at[0])
        launch_gathers(0, 0)
        pltpu.async_copy(sslice(1), src_v.at[1], sidx[1])
        pltpu.async_copy(dslice(1), dst_v.at[1], sidx[1])

        def step(i, _):
            for b in range(2):
                ch = 2 * i + b
                nxt = 1 - b

                @pl.when(ch + 1 < NCH_MP)
                def _():
                    pltpu.make_async_copy(sslice(ch + 1), src_v.at[nxt],
                                          sidx[nxt]).wait()
                    pltpu.make_async_copy(dslice(ch + 1), dst_v.at[nxt],
                                          sidx[nxt]).wait()
                    launch_gathers(ch + 1, nxt)

                wait_gathers(ch, b)
                compute_scatter(b)

                @pl.when(ch + 2 < NCH_MP)
                def _():
                    pltpu.async_copy(sslice(ch + 2), src_v.at[b], sidx[b])
                    pltpu.async_copy(dslice(ch + 2), dst_v.at[b], sidx[b])
            return 0
        lax.fori_loop(0, NCH_MP // 2, step, 0)

        plsc.subcore_barrier()
        pltpu.sync_copy(acc_s.at[pl.ds(s * NRT, NRT)],
                        acc_o.at[c, p, pl.ds(s * NRT, NRT)])
        plsc.subcore_barrier()


_msgpass = pl.kernel(
    _msgpass_body, mesh=_MESH,
    out_type=jax.ShapeDtypeStruct((NC, NP, N_ACC, 128), jnp.float32),
    scratch_types=[
        pltpu.VMEM((2, CM), jnp.int32),
        pltpu.VMEM((2, CM), jnp.int32),
        pltpu.VMEM((2, CM, 128), jnp.float32),
        pltpu.VMEM((2, CM, 128), jnp.float32),
        pltpu.VMEM((2, CM, 64), jnp.float32),
        pltpu.VMEM((2, CM, 128), jnp.float32),
        pltpu.VMEM_SHARED((N_ACC, 128), jnp.float32),
        pltpu.SemaphoreType.DMA,
        pltpu.SemaphoreType.DMA,
        pltpu.SemaphoreType.DMA,
        pltpu.SemaphoreType.DMA,
        pltpu.SemaphoreType.DMA,
        pltpu.SemaphoreType.DMA,
        pltpu.SemaphoreType.DMA,
        pltpu.SemaphoreType.DMA,
    ])


def _embed_body(x_ref, w_ref, b_ref, s_ref, bl_ref, o_ref):
    v = jnp.dot(x_ref[...], w_ref[...], preferred_element_type=jnp.float32)
    v = _silu(v + b_ref[...])
    o_ref[...] = _ln(v, s_ref[...], bl_ref[...])


def _encoder_body(xg0_ref, xg1_ref, r_ref, w1a_ref, w1b_ref, w1c_ref, b1_ref,
                  w2_ref, b2_ref, s_ref, bl_ref, o_ref):
    r = r_ref[...]
    dist = jnp.sqrt(jnp.sum(r * r, axis=-1, keepdims=True))
    centers = jax.lax.broadcasted_iota(jnp.int32, (1, BINS), 1).astype(jnp.float32) * (8.0 / (BINS - 1))
    width = 8.0 / BINS
    rbf = jnp.exp(-((dist - centers) ** 2) / (width ** 2))
    v = (jnp.dot(xg0_ref[0], w1a_ref[...], preferred_element_type=jnp.float32)
         + jnp.dot(xg1_ref[0], w1b_ref[...], preferred_element_type=jnp.float32)
         + jnp.dot(rbf, w1c_ref[...], preferred_element_type=jnp.float32))
    v = _silu(v + b1_ref[...])
    v = _silu(jnp.dot(v, w2_ref[...], preferred_element_type=jnp.float32) + b2_ref[...])
    o_ref[...] = _ln(v, s_ref[...], bl_ref[...])


def _tables_body(h_ref, wa_ref, ba_ref, wb_ref, bb_ref, wm_ref, bm_ref,
                 ws_ref, bs_ref, am0_ref, am1_ref, am2_ref, am3_ref,
                 bt0_ref, bt1_ref, hs_ref):
    h = h_ref[...]
    hA = jnp.dot(h, wa_ref[...], preferred_element_type=jnp.float32) + ba_ref[...]
    hB = jnp.dot(h, wb_ref[...], preferred_element_type=jnp.float32) + bb_ref[...]
    hM = jnp.dot(h, wm_ref[...], preferred_element_type=jnp.float32) + bm_ref[...]
    hS = jnp.dot(h, ws_ref[...], preferred_element_type=jnp.float32) + bs_ref[...]
    for p, am_ref in enumerate([am0_ref, am1_ref, am2_ref, am3_ref]):
        am_ref[...] = jnp.concatenate(
            [hA[:, 64 * p:64 * p + 64], hM[:, 64 * p:64 * p + 64]], axis=-1)
    bt0_ref[...] = hB[:, :128]
    bt1_ref[...] = hB[:, 128:]
    hs_ref[...] = hS


def _ec_body(ef_ref, w_ref, b_ref, e0_ref, e1_ref, e2_ref, e3_ref):
    v = jnp.dot(ef_ref[...], w_ref[...],
                preferred_element_type=jnp.float32) + b_ref[...]
    for p, o_ref in enumerate([e0_ref, e1_ref, e2_ref, e3_ref]):
        o_ref[...] = v[:, 64 * p:64 * p + 64]


def _update_body(h_ref, hs_ref, acc_ref, s_ref, bl_ref, o_ref):
    num = jnp.concatenate(
        [acc_ref[0, p, :, :64] + acc_ref[1, p, :, :64] for p in range(NP)], axis=-1)
    den = jnp.concatenate(
        [acc_ref[0, p, :, 64:] + acc_ref[1, p, :, 64:] for p in range(NP)], axis=-1)
    v = _silu(hs_ref[...] + num / (den + 1e-6))
    o_ref[...] = _ln(v + h_ref[...], s_ref[...], bl_ref[...])


def _head_body(ef_ref, hs_ref, hd_ref, w1a_ref, w1b_ref, w1c_ref, b1_ref,
               w2_ref, b2_ref, w3_ref, b3_ref, w4_ref, b4_ref, o_ref):
    def unpk(w):
        lo = jax.lax.bitcast_convert_type(w << 16, jnp.float32)
        hi = jax.lax.bitcast_convert_type(w & jnp.int32(-65536), jnp.float32)
        return jnp.concatenate([lo, hi], axis=-1)
    hs = unpk(hs_ref[0])
    hd = unpk(hd_ref[0])
    v = (jnp.dot(ef_ref[...], w1a_ref[...], preferred_element_type=jnp.float32)
         + jnp.dot(hs, w1b_ref[...], preferred_element_type=jnp.float32)
         + jnp.dot(hd, w1c_ref[...], preferred_element_type=jnp.float32))
    v = _silu(v + b1_ref[...])
    v = _silu(jnp.dot(v, w2_ref[...], preferred_element_type=jnp.float32) + b2_ref[...])
    v = _silu(jnp.dot(v, w3_ref[...], preferred_element_type=jnp.float32) + b3_ref[...])
    v = jnp.dot(v, w4_ref[...], preferred_element_type=jnp.float32) + b4_ref[...]
    o_ref[...] = jax.nn.sigmoid(v)


def _row_spec(blk, width):
    return pl.BlockSpec((blk, width), lambda i: (i, 0))


def _full_spec(shape):
    return pl.BlockSpec(shape, lambda i: tuple(0 for _ in shape))


def _tc_call(body, grid, in_specs, out_specs, out_shape, args):
    return pl.pallas_call(
        body, grid=(grid,), in_specs=in_specs, out_specs=out_specs,
        out_shape=out_shape)(*args)


@jax.jit
def _forward_impl(x, r, params, edge_index):
    pe = params["atom_emb"]
    h = _tc_call(
        _embed_body, N // BN,
        [_row_spec(BN, F), _full_spec((F, G)), _full_spec((1, G)),
         _full_spec((1, G)), _full_spec((1, G))],
        _row_spec(BN, G), jax.ShapeDtypeStruct((N, G), jnp.float32),
        (x, pe["W"], pe["b"].reshape(1, G),
         params["atom_ln"]["s"].reshape(1, G), params["atom_ln"]["b"].reshape(1, G)))

    src = edge_index[0]
    dst = edge_index[1]
    xg = _gather2_x(x, src, dst)
    r8 = jnp.pad(r, ((0, 0), (0, 5)))
    p1, p2 = params["edge1"], params["edge2"]
    ef = _tc_call(
        _encoder_body, E // BE,
        [pl.BlockSpec((1, BE, F), lambda i: (0, i, 0)),
         pl.BlockSpec((1, BE, F), lambda i: (1, i, 0)),
         _row_spec(BE, 8),
         _full_spec((F, EH)), _full_spec((F, EH)), _full_spec((BINS, EH)),
         _full_spec((1, EH)), _full_spec((EH, EH)), _full_spec((1, EH)),
         _full_spec((1, EH)), _full_spec((1, EH))],
        _row_spec(BE, EH), jax.ShapeDtypeStruct((E, EH), jnp.float32),
        (xg, xg, r8, p1["W"][:F], p1["W"][F:2 * F], p1["W"][2 * F:],
         p1["b"].reshape(1, EH), p2["W"], p2["b"].reshape(1, EH),
         params["edge_ln"]["s"].reshape(1, EH), params["edge_ln"]["b"].reshape(1, EH)))

    zeros_h = jnp.zeros((NRT, 128), jnp.float32)

    for lp, lnp in zip(params["gnn"], params["lns"]):
        tbl = pl.pallas_call(
            _tables_body, grid=(N // BN,),
            in_specs=[_row_spec(BN, G),
                      _full_spec((G, G)), _full_spec((1, G)),
                      _full_spec((G, G)), _full_spec((1, G)),
                      _full_spec((G, G)), _full_spec((1, G)),
                      _full_spec((G, G)), _full_spec((1, G))],
            out_specs=[_row_spec(BN, 128)] * 6 + [_row_spec(BN, G)],
            out_shape=[jax.ShapeDtypeStruct((N, 128), jnp.float32)] * 6
                      + [jax.ShapeDtypeStruct((N, G), jnp.float32)],
        )(h, lp["A"]["W"], lp["A"]["b"].reshape(1, G),
          lp["B"]["W"], lp["B"]["b"].reshape(1, G),
          lp["Wm"]["W"], lp["Wm"]["b"].reshape(1, G),
          lp["Ws"]["W"], lp["Ws"]["b"].reshape(1, G))
        am = tbl[:4]
        bt = tbl[4:6]
        hs = tbl[6]

        ec = _tc_call(
            _ec_body, E // BE,
            [_row_spec(BE, EH), _full_spec((EH, G)), _full_spec((1, G))],
            [_row_spec(BE, 64)] * 4,
            [jax.ShapeDtypeStruct((E, 64), jnp.float32)] * 4,
            (ef, lp["C"]["W"], lp["C"]["b"].reshape(1, G)))

        acc = _msgpass(*am, *bt, *ec, src, dst, zeros_h)

        h = _tc_call(
            _update_body, N // BN,
            [_row_spec(BN, G), _row_spec(BN, G),
             pl.BlockSpec((NC, NP, BN, 128), lambda i: (0, 0, i, 0)),
             _full_spec((1, G)), _full_spec((1, G))],
            _row_spec(BN, G), jax.ShapeDtypeStruct((N, G), jnp.float32),
            (h, hs, acc,
             lnp["s"].reshape(1, G), lnp["b"].reshape(1, G)))

    h2 = jax.lax.bitcast_convert_type(
        h.astype(jnp.bfloat16).reshape(N, G // 2, 2), jnp.int32)
    hg = _gather2_h(h2, src, dst)
    ordr = np.concatenate([np.arange(0, G, 2), np.arange(1, G, 2)])
    w1 = params["p1"]["W"]
    o = _tc_call(
        _head_body, E // BE,
        [_row_spec(BE, EH),
         pl.BlockSpec((1, BE, G // 2), lambda i: (0, i, 0)),
         pl.BlockSpec((1, BE, G // 2), lambda i: (1, i, 0)),
         _full_spec((EH, 256)), _full_spec((G, 256)), _full_spec((G, 256)),
         _full_spec((1, 256)), _full_spec((256, 128)), _full_spec((1, 128)),
         _full_spec((128, 64)), _full_spec((1, 64)), _full_spec((64, 1)),
         _full_spec((1, 1))],
        _row_spec(BE, 1), jax.ShapeDtypeStruct((E, 1), jnp.float32),
        (ef, hg, hg, w1[:EH], jnp.take(w1[EH:EH + G], ordr, axis=0),
         jnp.take(w1[EH + G:], ordr, axis=0),
         params["p1"]["b"].reshape(1, 256),
         params["p2"]["W"], params["p2"]["b"].reshape(1, 128),
         params["p3"]["W"], params["p3"]["b"].reshape(1, 64),
         params["p4"]["W"], params["p4"]["b"].reshape(1, 1)))
    return o


def kernel(x, r, params, edge_index):
    return _forward_impl(x, r, params, edge_index)


# speedup vs baseline: 3.1705x; 1.0170x over previous
import functools
import numpy as np
import jax
import jax.numpy as jnp
from jax import lax
from jax.experimental import pallas as pl
from jax.experimental.pallas import tpu as pltpu
from jax.experimental.pallas import tpu_sc as plsc

N, E, F, G, EH, BINS, L = 10000, 320000, 128, 256, 128, 40, 4
BN = 1000
BE = 512

NC, NS = 2, 16
NW = NC * NS
CE = 80
EPT = E // NW
NCH = EPT // CE
EPC = E // NC
CM = 40
NCH_MP = EPC // NS // CM
N_ACC = 10240
NRT = N_ACC // NS
NP = 4

_MESH = plsc.VectorSubcoreMesh(core_axis_name="c", subcore_axis_name="s")


def _silu(v):
    return v * jax.nn.sigmoid(v)


def _ln(v, s, b, eps=1e-5):
    mu = jnp.mean(v, axis=-1, keepdims=True)
    var = jnp.mean((v - mu) ** 2, axis=-1, keepdims=True)
    return (v - mu) * jax.lax.rsqrt(var + eps) * s + b


def _make_gather2(D, dtype=jnp.float32, split=1):
    rshape = (CE, D) if split == 1 else (CE, split, D // split)
    oshape = (2, E, D) if split == 1 else (2, E, split, D // split)

    def body(table, idx0, idx1, out, idx_v, rows_v, si0, si1, sg0, sg1):
        c = lax.axis_index("c")
        s = lax.axis_index("s")
        w = c * NS + s
        sidx = [si0, si1]
        sgat = [sg0, sg1]
        for k, idx in enumerate([idx0, idx1]):
            base0 = w * EPT

            def islice(ch):
                return idx.at[pl.ds(base0 + ch * CE, CE)]

            pltpu.sync_copy(islice(0), idx_v.at[0])
            pltpu.async_copy(table.at[idx_v.at[0]], rows_v.at[0], sgat[0])
            pltpu.async_copy(islice(1), idx_v.at[1], sidx[1])

            def step(i, _):
                for b in range(2):
                    ch = 2 * i + b
                    nxt = 1 - b
                    pltpu.make_async_copy(islice(ch + 1), idx_v.at[nxt],
                                          sidx[nxt]).wait()
                    pltpu.async_copy(table.at[idx_v.at[nxt]],
                                     rows_v.at[nxt], sgat[nxt])
                    pltpu.make_async_copy(table.at[idx_v.at[b]],
                                          rows_v.at[b], sgat[b]).wait()
                    pltpu.sync_copy(rows_v.at[b],
                                    out.at[k, pl.ds(base0 + ch * CE, CE)])

                    @pl.when(ch + 2 < NCH)
                    def _():
                        pltpu.async_copy(islice(ch + 2), idx_v.at[b], sidx[b])
                return 0
            lax.fori_loop(0, (NCH - 1) // 2, step, 0)
            pltpu.make_async_copy(table.at[idx_v.at[0]], rows_v.at[0],
                                  sgat[0]).wait()
            pltpu.sync_copy(rows_v.at[0],
                            out.at[k, pl.ds(base0 + (NCH - 1) * CE, CE)])

    return pl.kernel(
        body, mesh=_MESH,
        out_type=jax.ShapeDtypeStruct(oshape, dtype),
        scratch_types=[
            pltpu.VMEM((2, CE), jnp.int32),
            pltpu.VMEM((2,) + rshape, dtype),
            pltpu.SemaphoreType.DMA,
            pltpu.SemaphoreType.DMA,
            pltpu.SemaphoreType.DMA,
            pltpu.SemaphoreType.DMA,
        ])


_gather2_x = _make_gather2(F)
_gather2_h = _make_gather2(G // 2, jnp.int32)


def _msgpass_body(am0, am1, am2, am3, bt0, bt1, ec0, ec1, ec2, ec3,
                  src_h, dst_h, zeros_h, acc_o,
                  src_v, dst_v, am_v, b_v, ec_v, ms_v, acc_s,
                  si0, si1, sa0, sa1, sb0, sb1, se0, se1):
    c = lax.axis_index("c")
    s = lax.axis_index("s")
    tile_base = c * EPC + s * (EPC // NS)
    ams = [am0, am1, am2, am3]
    bts = [bt0, bt1]
    ecs = [ec0, ec1, ec2, ec3]
    sidx = [si0, si1]
    sam = [sa0, sa1]
    sbt = [sb0, sb1]
    sec = [se0, se1]
    for p in range(NP):
        boff = (p % 2) * 64
        amt = ams[p]
        btt = bts[p // 2]
        ect = ecs[p]

        def sslice(ch):
            return src_h.at[pl.ds(tile_base + ch * CM, CM)]

        def dslice(ch):
            return dst_h.at[pl.ds(tile_base + ch * CM, CM)]

        def eslice(ch):
            return ect.at[pl.ds(tile_base + ch * CM, CM)]

        def launch_gathers(ch, b):
            pltpu.async_copy(amt.at[src_v.at[b]], am_v.at[b], sam[b])
            pltpu.async_copy(btt.at[dst_v.at[b]], b_v.at[b], sbt[b])
            pltpu.async_copy(eslice(ch), ec_v.at[b], sec[b])

        def wait_gathers(ch, b):
            pltpu.make_async_copy(amt.at[src_v.at[b]], am_v.at[b],
                                  sam[b]).wait()
            pltpu.make_async_copy(btt.at[dst_v.at[b]], b_v.at[b],
                                  sbt[b]).wait()
            pltpu.make_async_copy(eslice(ch), ec_v.at[b], sec[b]).wait()

        def compute_scatter(b):
            amb = am_v.at[b]
            bvb = b_v.at[b]
            ecb = ec_v.at[b]
            msb = ms_v.at[b]

            def edge(e, _):
                for g in range(2):
                    cw = ecb[e, pl.ds(16 * g, 16)]
                    c_lo = plsc.bitcast(cw << 16, jnp.float32)
                    c_hi = plsc.bitcast(cw & jnp.int32(-65536), jnp.float32)
                    for half, cc in enumerate([c_lo, c_hi]):
                        j = g + 2 * half
                        a = amb[e, pl.ds(16 * j, 16)]
                        m = amb[e, pl.ds(64 + 16 * j, 16)]
                        bb = bvb[e, pl.ds(boff + 16 * j, 16)]
                        sg = 1.0 / (1.0 + jnp.exp(-(a + bb + cc)))
                        msb[e, pl.ds(16 * j, 16)] = sg * m
                        msb[e, pl.ds(64 + 16 * j, 16)] = sg
                return 0
            lax.fori_loop(0, CM, edge, 0)
            pltpu.sync_copy(msb, acc_s.at[dst_v.at[b]], add=True)

        pltpu.sync_copy(zeros_h, acc_s.at[pl.ds(s * NRT, NRT)])
        plsc.subcore_barrier()

        pltpu.sync_copy(sslice(0), src_v.at[0])
        pltpu.sync_copy(dslice(0), dst_v.at[0])
        launch_gathers(0, 0)
        pltpu.async_copy(sslice(1), src_v.at[1], sidx[1])
        pltpu.async_copy(dslice(1), dst_v.at[1], sidx[1])

        def step(i, _):
            for b in range(2):
                ch = 2 * i + b
                nxt = 1 - b

                @pl.when(ch + 1 < NCH_MP)
                def _():
                    pltpu.make_async_copy(sslice(ch + 1), src_v.at[nxt],
                                          sidx[nxt]).wait()
                    pltpu.make_async_copy(dslice(ch + 1), dst_v.at[nxt],
                                          sidx[nxt]).wait()
                    launch_gathers(ch + 1, nxt)

                wait_gathers(ch, b)
                compute_scatter(b)

                @pl.when(ch + 2 < NCH_MP)
                def _():
                    pltpu.async_copy(sslice(ch + 2), src_v.at[b], sidx[b])
                    pltpu.async_copy(dslice(ch + 2), dst_v.at[b], sidx[b])
            return 0
        lax.fori_loop(0, NCH_MP // 2, step, 0)

        plsc.subcore_barrier()
        pltpu.sync_copy(acc_s.at[pl.ds(s * NRT, NRT)],
                        acc_o.at[c, p, pl.ds(s * NRT, NRT)])
        plsc.subcore_barrier()


_msgpass = pl.kernel(
    _msgpass_body, mesh=_MESH,
    out_type=jax.ShapeDtypeStruct((NC, NP, N_ACC, 128), jnp.float32),
    compiler_params=pltpu.CompilerParams(needs_layout_passes=False),
    scratch_types=[
        pltpu.VMEM((2, CM), jnp.int32),
        pltpu.VMEM((2, CM), jnp.int32),
        pltpu.VMEM((2, CM, 128), jnp.float32),
        pltpu.VMEM((2, CM, 128), jnp.float32),
        pltpu.VMEM((2, CM, 32), jnp.int32),
        pltpu.VMEM((2, CM, 128), jnp.float32),
        pltpu.VMEM_SHARED((N_ACC, 128), jnp.float32),
        pltpu.SemaphoreType.DMA,
        pltpu.SemaphoreType.DMA,
        pltpu.SemaphoreType.DMA,
        pltpu.SemaphoreType.DMA,
        pltpu.SemaphoreType.DMA,
        pltpu.SemaphoreType.DMA,
        pltpu.SemaphoreType.DMA,
        pltpu.SemaphoreType.DMA,
    ])


def _embed_body(x_ref, w_ref, b_ref, s_ref, bl_ref, o_ref):
    v = jnp.dot(x_ref[...], w_ref[...], preferred_element_type=jnp.float32)
    v = _silu(v + b_ref[...])
    o_ref[...] = _ln(v, s_ref[...], bl_ref[...])


def _encoder_body(xg0_ref, xg1_ref, r_ref, w1a_ref, w1b_ref, w1c_ref, b1_ref,
                  w2_ref, b2_ref, s_ref, bl_ref, o_ref):
    r = r_ref[...]
    dist = jnp.sqrt(jnp.sum(r * r, axis=-1, keepdims=True))
    centers = jax.lax.broadcasted_iota(jnp.int32, (1, BINS), 1).astype(jnp.float32) * (8.0 / (BINS - 1))
    width = 8.0 / BINS
    rbf = jnp.exp(-((dist - centers) ** 2) / (width ** 2))
    v = (jnp.dot(xg0_ref[0], w1a_ref[...], preferred_element_type=jnp.float32)
         + jnp.dot(xg1_ref[0], w1b_ref[...], preferred_element_type=jnp.float32)
         + jnp.dot(rbf, w1c_ref[...], preferred_element_type=jnp.float32))
    v = _silu(v + b1_ref[...])
    v = _silu(jnp.dot(v, w2_ref[...], preferred_element_type=jnp.float32) + b2_ref[...])
    o_ref[...] = _ln(v, s_ref[...], bl_ref[...])


def _tables_body(h_ref, wa_ref, ba_ref, wb_ref, bb_ref, wm_ref, bm_ref,
                 ws_ref, bs_ref, am0_ref, am1_ref, am2_ref, am3_ref,
                 bt0_ref, bt1_ref, hs_ref):
    h = h_ref[...]
    hA = jnp.dot(h, wa_ref[...], preferred_element_type=jnp.float32) + ba_ref[...]
    hB = jnp.dot(h, wb_ref[...], preferred_element_type=jnp.float32) + bb_ref[...]
    hM = jnp.dot(h, wm_ref[...], preferred_element_type=jnp.float32) + bm_ref[...]
    hS = jnp.dot(h, ws_ref[...], preferred_element_type=jnp.float32) + bs_ref[...]
    for p, am_ref in enumerate([am0_ref, am1_ref, am2_ref, am3_ref]):
        am_ref[...] = jnp.concatenate(
            [hA[:, 64 * p:64 * p + 64], hM[:, 64 * p:64 * p + 64]], axis=-1)
    bt0_ref[...] = hB[:, :128]
    bt1_ref[...] = hB[:, 128:]
    hs_ref[...] = hS


def _ec_body(ef_ref, w_ref, b_ref, e0_ref, e1_ref, e2_ref, e3_ref):
    v = jnp.dot(ef_ref[...], w_ref[...],
                preferred_element_type=jnp.float32) + b_ref[...]
    fi = jax.lax.bitcast_convert_type(v[:, :128], jnp.int32)
    si = jax.lax.bitcast_convert_type(v[:, 128:], jnp.int32)
    rf = fi + 0x7FFF + ((fi >> 16) & 1)
    rs = si + 0x7FFF + ((si >> 16) & 1)
    w = ((rf >> 16) & 0xFFFF) | (rs & jnp.int32(-65536))
    for p, o_ref in enumerate([e0_ref, e1_ref, e2_ref, e3_ref]):
        o_ref[...] = w[:, 32 * p:32 * p + 32]


def _update_body(h_ref, hs_ref, acc_ref, s_ref, bl_ref, o_ref):
    num = jnp.concatenate(
        [acc_ref[0, p, :, :64] + acc_ref[1, p, :, :64] for p in range(NP)], axis=-1)
    den = jnp.concatenate(
        [acc_ref[0, p, :, 64:] + acc_ref[1, p, :, 64:] for p in range(NP)], axis=-1)
    v = _silu(hs_ref[...] + num / (den + 1e-6))
    o_ref[...] = _ln(v + h_ref[...], s_ref[...], bl_ref[...])


def _head_body(ef_ref, hs_ref, hd_ref, w1a_ref, w1b_ref, w1c_ref, b1_ref,
               w2_ref, b2_ref, w3_ref, b3_ref, w4_ref, b4_ref, o_ref):
    def unpk(w):
        lo = jax.lax.bitcast_convert_type(w << 16, jnp.float32)
        hi = jax.lax.bitcast_convert_type(w & jnp.int32(-65536), jnp.float32)
        return jnp.concatenate([lo, hi], axis=-1)
    hs = unpk(hs_ref[0])
    hd = unpk(hd_ref[0])
    v = (jnp.dot(ef_ref[...], w1a_ref[...], preferred_element_type=jnp.float32)
         + jnp.dot(hs, w1b_ref[...], preferred_element_type=jnp.float32)
         + jnp.dot(hd, w1c_ref[...], preferred_element_type=jnp.float32))
    v = _silu(v + b1_ref[...])
    v = _silu(jnp.dot(v, w2_ref[...], preferred_element_type=jnp.float32) + b2_ref[...])
    v = _silu(jnp.dot(v, w3_ref[...], preferred_element_type=jnp.float32) + b3_ref[...])
    v = jnp.dot(v, w4_ref[...], preferred_element_type=jnp.float32) + b4_ref[...]
    o_ref[...] = jax.nn.sigmoid(v)


def _row_spec(blk, width):
    return pl.BlockSpec((blk, width), lambda i: (i, 0))


def _full_spec(shape):
    return pl.BlockSpec(shape, lambda i: tuple(0 for _ in shape))


def _tc_call(body, grid, in_specs, out_specs, out_shape, args):
    return pl.pallas_call(
        body, grid=(grid,), in_specs=in_specs, out_specs=out_specs,
        out_shape=out_shape)(*args)


@jax.jit
def _forward_impl(x, r, params, edge_index):
    pe = params["atom_emb"]
    h = _tc_call(
        _embed_body, N // BN,
        [_row_spec(BN, F), _full_spec((F, G)), _full_spec((1, G)),
         _full_spec((1, G)), _full_spec((1, G))],
        _row_spec(BN, G), jax.ShapeDtypeStruct((N, G), jnp.float32),
        (x, pe["W"], pe["b"].reshape(1, G),
         params["atom_ln"]["s"].reshape(1, G), params["atom_ln"]["b"].reshape(1, G)))

    src = edge_index[0]
    dst = edge_index[1]
    xg = _gather2_x(x, src, dst)
    r8 = jnp.pad(r, ((0, 0), (0, 5)))
    p1, p2 = params["edge1"], params["edge2"]
    ef = _tc_call(
        _encoder_body, E // BE,
        [pl.BlockSpec((1, BE, F), lambda i: (0, i, 0)),
         pl.BlockSpec((1, BE, F), lambda i: (1, i, 0)),
         _row_spec(BE, 8),
         _full_spec((F, EH)), _full_spec((F, EH)), _full_spec((BINS, EH)),
         _full_spec((1, EH)), _full_spec((EH, EH)), _full_spec((1, EH)),
         _full_spec((1, EH)), _full_spec((1, EH))],
        _row_spec(BE, EH), jax.ShapeDtypeStruct((E, EH), jnp.float32),
        (xg, xg, r8, p1["W"][:F], p1["W"][F:2 * F], p1["W"][2 * F:],
         p1["b"].reshape(1, EH), p2["W"], p2["b"].reshape(1, EH),
         params["edge_ln"]["s"].reshape(1, EH), params["edge_ln"]["b"].reshape(1, EH)))

    zeros_h = jnp.zeros((NRT, 128), jnp.float32)

    for lp, lnp in zip(params["gnn"], params["lns"]):
        tbl = pl.pallas_call(
            _tables_body, grid=(N // BN,),
            in_specs=[_row_spec(BN, G),
                      _full_spec((G, G)), _full_spec((1, G)),
                      _full_spec((G, G)), _full_spec((1, G)),
                      _full_spec((G, G)), _full_spec((1, G)),
                      _full_spec((G, G)), _full_spec((1, G))],
            out_specs=[_row_spec(BN, 128)] * 6 + [_row_spec(BN, G)],
            out_shape=[jax.ShapeDtypeStruct((N, 128), jnp.float32)] * 6
                      + [jax.ShapeDtypeStruct((N, G), jnp.float32)],
        )(h, lp["A"]["W"], lp["A"]["b"].reshape(1, G),
          lp["B"]["W"], lp["B"]["b"].reshape(1, G),
          lp["Wm"]["W"], lp["Wm"]["b"].reshape(1, G),
          lp["Ws"]["W"], lp["Ws"]["b"].reshape(1, G))
        am = tbl[:4]
        bt = tbl[4:6]
        hs = tbl[6]

        k = np.arange(128)
        cperm = np.concatenate([64 * (k // 32) + k % 32,
                                64 * (k // 32) + 32 + k % 32])
        w_c = jnp.take(lp["C"]["W"], cperm, axis=1)
        b_c = jnp.take(lp["C"]["b"], cperm)
        ec = _tc_call(
            _ec_body, E // BE,
            [_row_spec(BE, EH), _full_spec((EH, G)), _full_spec((1, G))],
            [_row_spec(BE, 32)] * 4,
            [jax.ShapeDtypeStruct((E, 32), jnp.int32)] * 4,
            (ef, w_c, b_c.reshape(1, G)))

        acc = _msgpass(*am, *bt, *ec, src, dst, zeros_h)

        h = _tc_call(
            _update_body, N // BN,
            [_row_spec(BN, G), _row_spec(BN, G),
             pl.BlockSpec((NC, NP, BN, 128), lambda i: (0, 0, i, 0)),
             _full_spec((1, G)), _full_spec((1, G))],
            _row_spec(BN, G), jax.ShapeDtypeStruct((N, G), jnp.float32),
            (h, hs, acc,
             lnp["s"].reshape(1, G), lnp["b"].reshape(1, G)))

    h2 = jax.lax.bitcast_convert_type(
        h.astype(jnp.bfloat16).reshape(N, G // 2, 2), jnp.int32)
    hg = _gather2_h(h2, src, dst)
    ordr = np.concatenate([np.arange(0, G, 2), np.arange(1, G, 2)])
    w1 = params["p1"]["W"]
    o = _tc_call(
        _head_body, E // BE,
        [_row_spec(BE, EH),
         pl.BlockSpec((1, BE, G // 2), lambda i: (0, i, 0)),
         pl.BlockSpec((1, BE, G // 2), lambda i: (1, i, 0)),
         _full_spec((EH, 256)), _full_spec((G, 256)), _full_spec((G, 256)),
         _full_spec((1, 256)), _full_spec((256, 128)), _full_spec((1, 128)),
         _full_spec((128, 64)), _full_spec((1, 64)), _full_spec((64, 1)),
         _full_spec((1, 1))],
        _row_spec(BE, 1), jax.ShapeDtypeStruct((E, 1), jnp.float32),
        (ef, hg, hg, w1[:EH], jnp.take(w1[EH:EH + G], ordr, axis=0),
         jnp.take(w1[EH + G:], ordr, axis=0),
         params["p1"]["b"].reshape(1, 256),
         params["p2"]["W"], params["p2"]["b"].reshape(1, 128),
         params["p3"]["W"], params["p3"]["b"].reshape(1, 64),
         params["p4"]["W"], params["p4"]["b"].reshape(1, 1)))
    return o


def kernel(x, r, params, edge_index):
    return _forward_impl(x, r, params, edge_index)


# speedup vs baseline: 3.1712x; 1.0002x over previous
import functools
import numpy as np
import jax
import jax.numpy as jnp
from jax import lax
from jax.experimental import pallas as pl
from jax.experimental.pallas import tpu as pltpu
from jax.experimental.pallas import tpu_sc as plsc

N, E, F, G, EH, BINS, L = 10000, 320000, 128, 256, 128, 40, 4
BN = 1000
BE = 512

NC, NS = 2, 16
NW = NC * NS
CE = 80
EPT = E // NW
NCH = EPT // CE
EPC = E // NC
CM = 40
NCH_MP = EPC // NS // CM
N_ACC = 10240
NRT = N_ACC // NS
NP = 4

_MESH = plsc.VectorSubcoreMesh(core_axis_name="c", subcore_axis_name="s")


def _silu(v):
    return v * jax.nn.sigmoid(v)


def _ln(v, s, b, eps=1e-5):
    mu = jnp.mean(v, axis=-1, keepdims=True)
    var = jnp.mean((v - mu) ** 2, axis=-1, keepdims=True)
    return (v - mu) * jax.lax.rsqrt(var + eps) * s + b


def _make_gather2(D, dtype=jnp.float32, split=1):
    rshape = (CE, D) if split == 1 else (CE, split, D // split)
    oshape = (2, E, D) if split == 1 else (2, E, split, D // split)

    def body(table, idx0, idx1, out, idx_v, rows_v, si0, si1, sg0, sg1):
        c = lax.axis_index("c")
        s = lax.axis_index("s")
        w = c * NS + s
        sidx = [si0, si1]
        sgat = [sg0, sg1]
        for k, idx in enumerate([idx0, idx1]):
            base0 = w * EPT

            def islice(ch):
                return idx.at[pl.ds(base0 + ch * CE, CE)]

            pltpu.sync_copy(islice(0), idx_v.at[0])
            pltpu.async_copy(table.at[idx_v.at[0]], rows_v.at[0], sgat[0])
            pltpu.async_copy(islice(1), idx_v.at[1], sidx[1])

            def step(i, _):
                for b in range(2):
                    ch = 2 * i + b
                    nxt = 1 - b
                    pltpu.make_async_copy(islice(ch + 1), idx_v.at[nxt],
                                          sidx[nxt]).wait()
                    pltpu.async_copy(table.at[idx_v.at[nxt]],
                                     rows_v.at[nxt], sgat[nxt])
                    pltpu.make_async_copy(table.at[idx_v.at[b]],
                                          rows_v.at[b], sgat[b]).wait()
                    pltpu.sync_copy(rows_v.at[b],
                                    out.at[k, pl.ds(base0 + ch * CE, CE)])

                    @pl.when(ch + 2 < NCH)
                    def _():
                        pltpu.async_copy(islice(ch + 2), idx_v.at[b], sidx[b])
                return 0
            lax.fori_loop(0, (NCH - 1) // 2, step, 0)
            pltpu.make_async_copy(table.at[idx_v.at[0]], rows_v.at[0],
                                  sgat[0]).wait()
            pltpu.sync_copy(rows_v.at[0],
                            out.at[k, pl.ds(base0 + (NCH - 1) * CE, CE)])

    return pl.kernel(
        body, mesh=_MESH,
        out_type=jax.ShapeDtypeStruct(oshape, dtype),
        scratch_types=[
            pltpu.VMEM((2, CE), jnp.int32),
            pltpu.VMEM((2,) + rshape, dtype),
            pltpu.SemaphoreType.DMA,
            pltpu.SemaphoreType.DMA,
            pltpu.SemaphoreType.DMA,
            pltpu.SemaphoreType.DMA,
        ])


_gather2_x = _make_gather2(F)
_gather2_h = _make_gather2(G // 2, jnp.int32)


def _msgpass_body(am0, am1, am2, am3, bt0, bt1, ec0, ec1, ec2, ec3,
                  src_h, dst_h, zeros_h, acc_o,
                  src_v, dst_v, am_v, b_v, ec_v, ms_v, acc_s,
                  si0, si1, sa0, sa1, sb0, sb1, se0, se1):
    c = lax.axis_index("c")
    s = lax.axis_index("s")
    tile_base = c * EPC + s * (EPC // NS)
    ams = [am0, am1, am2, am3]
    bts = [bt0, bt1]
    ecs = [ec0, ec1, ec2, ec3]
    sidx = [si0, si1]
    sam = [sa0, sa1]
    sbt = [sb0, sb1]
    sec = [se0, se1]
    for p in range(NP):
        boff = (p % 2) * 64
        amt = ams[p]
        btt = bts[p // 2]
        ect = ecs[p]

        def sslice(ch):
            return src_h.at[pl.ds(tile_base + ch * CM, CM)]

        def dslice(ch):
            return dst_h.at[pl.ds(tile_base + ch * CM, CM)]

        def eslice(ch):
            return ect.at[pl.ds(tile_base + ch * CM, CM)]

        def launch_gathers(ch, b):
            pltpu.async_copy(amt.at[src_v.at[b]], am_v.at[b], sam[b])
            pltpu.async_copy(btt.at[dst_v.at[b]], b_v.at[b], sbt[b])
            pltpu.async_copy(eslice(ch), ec_v.at[b], sec[b])

        def wait_gathers(ch, b):
            pltpu.make_async_copy(amt.at[src_v.at[b]], am_v.at[b],
                                  sam[b]).wait()
            pltpu.make_async_copy(btt.at[dst_v.at[b]], b_v.at[b],
                                  sbt[b]).wait()
            pltpu.make_async_copy(eslice(ch), ec_v.at[b], sec[b]).wait()

        def compute_scatter(b):
            amb = am_v.at[b]
            bvb = b_v.at[b]
            ecb = ec_v.at[b]
            msb = ms_v.at[b]

            def edge(e, _):
                for g in range(2):
                    cw = ecb[e, pl.ds(16 * g, 16)]
                    c_lo = plsc.bitcast(cw << 16, jnp.float32)
                    c_hi = plsc.bitcast(cw & jnp.int32(-65536), jnp.float32)
                    for half, cc in enumerate([c_lo, c_hi]):
                        j = g + 2 * half
                        a = amb[e, pl.ds(16 * j, 16)]
                        m = amb[e, pl.ds(64 + 16 * j, 16)]
                        bb = bvb[e, pl.ds(boff + 16 * j, 16)]
                        sg = 1.0 / (1.0 + jnp.exp(-(a + bb + cc)))
                        msb[e, pl.ds(16 * j, 16)] = sg * m
                        msb[e, pl.ds(64 + 16 * j, 16)] = sg
                return 0
            lax.fori_loop(0, CM, edge, 0)
            pltpu.sync_copy(msb, acc_s.at[dst_v.at[b]], add=True)

        pltpu.sync_copy(zeros_h, acc_s.at[pl.ds(s * NRT, NRT)])
        plsc.subcore_barrier()

        pltpu.sync_copy(sslice(0), src_v.at[0])
        pltpu.sync_copy(dslice(0), dst_v.at[0])
        launch_gathers(0, 0)
        pltpu.async_copy(sslice(1), src_v.at[1], sidx[1])
        pltpu.async_copy(dslice(1), dst_v.at[1], sidx[1])

        def step(i, _):
            for b in range(2):
                ch = 2 * i + b
                nxt = 1 - b

                @pl.when(ch + 1 < NCH_MP)
                def _():
                    pltpu.make_async_copy(sslice(ch + 1), src_v.at[nxt],
                                          sidx[nxt]).wait()
                    pltpu.make_async_copy(dslice(ch + 1), dst_v.at[nxt],
                                          sidx[nxt]).wait()
                    launch_gathers(ch + 1, nxt)

                wait_gathers(ch, b)
                compute_scatter(b)

                @pl.when(ch + 2 < NCH_MP)
                def _():
                    pltpu.async_copy(sslice(ch + 2), src_v.at[b], sidx[b])
                    pltpu.async_copy(dslice(ch + 2), dst_v.at[b], sidx[b])
            return 0
        lax.fori_loop(0, NCH_MP // 2, step, 0)

        plsc.subcore_barrier()
        pltpu.sync_copy(acc_s.at[pl.ds(s * NRT, NRT)],
                        acc_o.at[c, p, pl.ds(s * NRT, NRT)])
        plsc.subcore_barrier()


_msgpass = pl.kernel(
    _msgpass_body, mesh=_MESH,
    out_type=jax.ShapeDtypeStruct((NC, NP, N_ACC, 128), jnp.float32),
    compiler_params=pltpu.CompilerParams(needs_layout_passes=False),
    scratch_types=[
        pltpu.VMEM((2, CM), jnp.int32),
        pltpu.VMEM((2, CM), jnp.int32),
        pltpu.VMEM((2, CM, 128), jnp.float32),
        pltpu.VMEM((2, CM, 128), jnp.float32),
        pltpu.VMEM((2, CM, 32), jnp.int32),
        pltpu.VMEM((2, CM, 128), jnp.float32),
        pltpu.VMEM_SHARED((N_ACC, 128), jnp.float32),
        pltpu.SemaphoreType.DMA,
        pltpu.SemaphoreType.DMA,
        pltpu.SemaphoreType.DMA,
        pltpu.SemaphoreType.DMA,
        pltpu.SemaphoreType.DMA,
        pltpu.SemaphoreType.DMA,
        pltpu.SemaphoreType.DMA,
        pltpu.SemaphoreType.DMA,
    ])


def _embed_body(x_ref, w_ref, b_ref, s_ref, bl_ref, o_ref):
    v = jnp.dot(x_ref[...], w_ref[...], preferred_element_type=jnp.float32)
    v = _silu(v + b_ref[...])
    o_ref[...] = _ln(v, s_ref[...], bl_ref[...])


def _encoder_body(xg0_ref, xg1_ref, r_ref, w1a_ref, w1b_ref, w1c_ref, b1_ref,
                  w2_ref, b2_ref, s_ref, bl_ref, o_ref):
    r = r_ref[...]
    dist = jnp.sqrt(jnp.sum(r * r, axis=-1, keepdims=True))
    centers = jax.lax.broadcasted_iota(jnp.int32, (1, BINS), 1).astype(jnp.float32) * (8.0 / (BINS - 1))
    width = 8.0 / BINS
    rbf = jnp.exp(-((dist - centers) ** 2) / (width ** 2))
    v = (jnp.dot(xg0_ref[0], w1a_ref[...], preferred_element_type=jnp.float32)
         + jnp.dot(xg1_ref[0], w1b_ref[...], preferred_element_type=jnp.float32)
         + jnp.dot(rbf, w1c_ref[...], preferred_element_type=jnp.float32))
    v = _silu(v + b1_ref[...])
    v = _silu(jnp.dot(v, w2_ref[...], preferred_element_type=jnp.float32) + b2_ref[...])
    o_ref[...] = _ln(v, s_ref[...], bl_ref[...])


def _tables_body(h_ref, wa_ref, ba_ref, wb_ref, bb_ref, wm_ref, bm_ref,
                 ws_ref, bs_ref, am0_ref, am1_ref, am2_ref, am3_ref,
                 bt0_ref, bt1_ref, hs_ref):
    h = h_ref[...]
    hA = jnp.dot(h, wa_ref[...], preferred_element_type=jnp.float32) + ba_ref[...]
    hB = jnp.dot(h, wb_ref[...], preferred_element_type=jnp.float32) + bb_ref[...]
    hM = jnp.dot(h, wm_ref[...], preferred_element_type=jnp.float32) + bm_ref[...]
    hS = jnp.dot(h, ws_ref[...], preferred_element_type=jnp.float32) + bs_ref[...]
    for p, am_ref in enumerate([am0_ref, am1_ref, am2_ref, am3_ref]):
        am_ref[...] = jnp.concatenate(
            [hA[:, 64 * p:64 * p + 64], hM[:, 64 * p:64 * p + 64]], axis=-1)
    bt0_ref[...] = hB[:, :128]
    bt1_ref[...] = hB[:, 128:]
    hs_ref[...] = hS


def _ec_body(ef_ref, w_ref, b_ref, e0_ref, e1_ref, e2_ref, e3_ref):
    v = jnp.dot(ef_ref[...], w_ref[...],
                preferred_element_type=jnp.float32) + b_ref[...]
    fi = jax.lax.bitcast_convert_type(v[:, :128], jnp.int32)
    si = jax.lax.bitcast_convert_type(v[:, 128:], jnp.int32)
    rf = fi + 0x7FFF + ((fi >> 16) & 1)
    rs = si + 0x7FFF + ((si >> 16) & 1)
    w = ((rf >> 16) & 0xFFFF) | (rs & jnp.int32(-65536))
    for p, o_ref in enumerate([e0_ref, e1_ref, e2_ref, e3_ref]):
        o_ref[...] = w[:, 32 * p:32 * p + 32]


def _update_body(h_ref, hs_ref, acc_ref, s_ref, bl_ref, o_ref):
    num = jnp.concatenate(
        [acc_ref[0, p, :, :64] + acc_ref[1, p, :, :64] for p in range(NP)], axis=-1)
    den = jnp.concatenate(
        [acc_ref[0, p, :, 64:] + acc_ref[1, p, :, 64:] for p in range(NP)], axis=-1)
    v = _silu(hs_ref[...] + num / (den + 1e-6))
    o_ref[...] = _ln(v + h_ref[...], s_ref[...], bl_ref[...])


def _head_body(ef_ref, hs_ref, hd_ref, w1a_ref, w1b_ref, w1c_ref, b1_ref,
               w2_ref, b2_ref, w3_ref, b3_ref, w4_ref, b4_ref, o_ref):
    def unpk(w):
        lo = jax.lax.bitcast_convert_type(w << 16, jnp.float32)
        hi = jax.lax.bitcast_convert_type(w & jnp.int32(-65536), jnp.float32)
        return jnp.concatenate([lo, hi], axis=-1)
    hs = unpk(hs_ref[0])
    hd = unpk(hd_ref[0])
    v = (jnp.dot(ef_ref[...], w1a_ref[...], preferred_element_type=jnp.float32)
         + jnp.dot(hs, w1b_ref[...], preferred_element_type=jnp.float32)
         + jnp.dot(hd, w1c_ref[...], preferred_element_type=jnp.float32))
    v = _silu(v + b1_ref[...])
    v = _silu(jnp.dot(v, w2_ref[...], preferred_element_type=jnp.float32) + b2_ref[...])
    v = _silu(jnp.dot(v, w3_ref[...], preferred_element_type=jnp.float32) + b3_ref[...])
    v = jnp.dot(v, w4_ref[...], preferred_element_type=jnp.float32) + b4_ref[...]
    o_ref[...] = jax.nn.sigmoid(v)


def _row_spec(blk, width):
    return pl.BlockSpec((blk, width), lambda i: (i, 0))


def _full_spec(shape):
    return pl.BlockSpec(shape, lambda i: tuple(0 for _ in shape))


def _tc_call(body, grid, in_specs, out_specs, out_shape, args):
    return pl.pallas_call(
        body, grid=(grid,), in_specs=in_specs, out_specs=out_specs,
        out_shape=out_shape)(*args)


@jax.jit
def _forward_impl(x, r, params, edge_index):
    pe = params["atom_emb"]
    h = _tc_call(
        _embed_body, N // BN,
        [_row_spec(BN, F), _full_spec((F, G)), _full_spec((1, G)),
         _full_spec((1, G)), _full_spec((1, G))],
        _row_spec(BN, G), jax.ShapeDtypeStruct((N, G), jnp.float32),
        (x, pe["W"], pe["b"].reshape(1, G),
         params["atom_ln"]["s"].reshape(1, G), params["atom_ln"]["b"].reshape(1, G)))

    src = edge_index[0]
    dst = edge_index[1]
    xg = _gather2_x(x, src, dst)
    r8 = jnp.pad(r, ((0, 0), (0, 5)))
    p1, p2 = params["edge1"], params["edge2"]
    ef = _tc_call(
        _encoder_body, E // BE,
        [pl.BlockSpec((1, BE, F), lambda i: (0, i, 0)),
         pl.BlockSpec((1, BE, F), lambda i: (1, i, 0)),
         _row_spec(BE, 8),
         _full_spec((F, EH)), _full_spec((F, EH)), _full_spec((BINS, EH)),
         _full_spec((1, EH)), _full_spec((EH, EH)), _full_spec((1, EH)),
         _full_spec((1, EH)), _full_spec((1, EH))],
        _row_spec(BE, EH), jax.ShapeDtypeStruct((E, EH), jnp.float32),
        (xg, xg, r8, p1["W"][:F], p1["W"][F:2 * F], p1["W"][2 * F:],
         p1["b"].reshape(1, EH), p2["W"], p2["b"].reshape(1, EH),
         params["edge_ln"]["s"].reshape(1, EH), params["edge_ln"]["b"].reshape(1, EH)))

    zeros_h = jnp.zeros((NRT, 128), jnp.float32)

    ec_all = []
    for lp in params["gnn"]:
        k = np.arange(128)
        cperm = np.concatenate([64 * (k // 32) + k % 32,
                                64 * (k // 32) + 32 + k % 32])
        w_c = jnp.take(lp["C"]["W"], cperm, axis=1)
        b_c = jnp.take(lp["C"]["b"], cperm)
        ec_all.append(_tc_call(
            _ec_body, E // BE,
            [_row_spec(BE, EH), _full_spec((EH, G)), _full_spec((1, G))],
            [_row_spec(BE, 32)] * 4,
            [jax.ShapeDtypeStruct((E, 32), jnp.int32)] * 4,
            (ef, w_c, b_c.reshape(1, G))))

    for ec, (lp, lnp) in zip(ec_all, zip(params["gnn"], params["lns"])):
        tbl = pl.pallas_call(
            _tables_body, grid=(N // BN,),
            in_specs=[_row_spec(BN, G),
                      _full_spec((G, G)), _full_spec((1, G)),
                      _full_spec((G, G)), _full_spec((1, G)),
                      _full_spec((G, G)), _full_spec((1, G)),
                      _full_spec((G, G)), _full_spec((1, G))],
            out_specs=[_row_spec(BN, 128)] * 6 + [_row_spec(BN, G)],
            out_shape=[jax.ShapeDtypeStruct((N, 128), jnp.float32)] * 6
                      + [jax.ShapeDtypeStruct((N, G), jnp.float32)],
        )(h, lp["A"]["W"], lp["A"]["b"].reshape(1, G),
          lp["B"]["W"], lp["B"]["b"].reshape(1, G),
          lp["Wm"]["W"], lp["Wm"]["b"].reshape(1, G),
          lp["Ws"]["W"], lp["Ws"]["b"].reshape(1, G))
        am = tbl[:4]
        bt = tbl[4:6]
        hs = tbl[6]

        acc = _msgpass(*am, *bt, *ec, src, dst, zeros_h)

        h = _tc_call(
            _update_body, N // BN,
            [_row_spec(BN, G), _row_spec(BN, G),
             pl.BlockSpec((NC, NP, BN, 128), lambda i: (0, 0, i, 0)),
             _full_spec((1, G)), _full_spec((1, G))],
            _row_spec(BN, G), jax.ShapeDtypeStruct((N, G), jnp.float32),
            (h, hs, acc,
             lnp["s"].reshape(1, G), lnp["b"].reshape(1, G)))

    h2 = jax.lax.bitcast_convert_type(
        h.astype(jnp.bfloat16).reshape(N, G // 2, 2), jnp.int32)
    hg = _gather2_h(h2, src, dst)
    ordr = np.concatenate([np.arange(0, G, 2), np.arange(1, G, 2)])
    w1 = params["p1"]["W"]
    o = _tc_call(
        _head_body, E // BE,
        [_row_spec(BE, EH),
         pl.BlockSpec((1, BE, G // 2), lambda i: (0, i, 0)),
         pl.BlockSpec((1, BE, G // 2), lambda i: (1, i, 0)),
         _full_spec((EH, 256)), _full_spec((G, 256)), _full_spec((G, 256)),
         _full_spec((1, 256)), _full_spec((256, 128)), _full_spec((1, 128)),
         _full_spec((128, 64)), _full_spec((1, 64)), _full_spec((64, 1)),
         _full_spec((1, 1))],
        _row_spec(BE, 1), jax.ShapeDtypeStruct((E, 1), jnp.float32),
        (ef, hg, hg, w1[:EH], jnp.take(w1[EH:EH + G], ordr, axis=0),
         jnp.take(w1[EH + G:], ordr, axis=0),
         params["p1"]["b"].reshape(1, 256),
         params["p2"]["W"], params["p2"]["b"].reshape(1, 128),
         params["p3"]["W"], params["p3"]["b"].reshape(1, 64),
         params["p4"]["W"], params["p4"]["b"].reshape(1, 1)))
    return o


def kernel(x, r, params, edge_index):
    return _forward_impl(x, r, params, edge_index)
